# baseline (speedup 1.0000x reference)
import functools

import jax
import jax.numpy as jnp
from jax import lax
from jax.experimental import pallas as pl
from jax.experimental.pallas import tpu as pltpu
from jax.experimental.pallas import tpu_sc as plsc

N_NODES = 10000
N_EDGES = 160000
C = 128
BE = 2000
BN = 2000

SC_NC = 2
SC_NS = 16
SC_NW = SC_NC * SC_NS
EPW = N_EDGES // SC_NW
CH = 128
NCH = EPW // CH
REM = EPW - NCH * CH
NPAD = 10240
NPA = NPAD // SC_NS


def _sc_mesh():
    return plsc.VectorSubcoreMesh(core_axis_name="c", subcore_axis_name="s",
                                  num_cores=SC_NC, num_subcores=SC_NS)


def _wid():
    return lax.axis_index("s") * SC_NC + lax.axis_index("c")


def _make_sc_gather(d, dt=jnp.float32):
    @functools.partial(
        pl.kernel,
        out_type=jax.ShapeDtypeStruct((N_EDGES, d), dt),
        mesh=_sc_mesh(),
        scratch_types=[pltpu.VMEM((EPW,), jnp.int32),
                       pltpu.VMEM((CH, d), dt),
                       pltpu.VMEM((CH, d), dt),
                       pltpu.SemaphoreType.DMA,
                       pltpu.SemaphoreType.DMA],
    )
    def k(table_hbm, idx_hbm, out_hbm, idx_v, rows0, rows1, sem0, sem1):
        base = _wid() * EPW
        pltpu.sync_copy(idx_hbm.at[pl.ds(base, EPW)], idx_v)

        def gath(j, buf, sem):
            return pltpu.async_copy(
                table_hbm.at[idx_v.at[pl.ds(j * CH, CH)]], buf, sem)

        def put(j, buf):
            pltpu.sync_copy(buf, out_hbm.at[pl.ds(base + j * CH, CH)])

        def drain(buf, sem):
            pltpu.make_async_copy(table_hbm.at[pl.ds(0, CH)], buf, sem).wait()

        gath(0, rows0, sem0)

        def body(t, _):
            j = t * 2
            gath(j + 1, rows1, sem1)
            drain(rows0, sem0)
            put(j, rows0)
            gath(j + 2, rows0, sem0)
            drain(rows1, sem1)
            put(j + 1, rows1)
            return _

        lax.fori_loop(0, (NCH - 1) // 2, body, 0)
        drain(rows0, sem0)
        put(NCH - 1, rows0)
        off = NCH * CH
        pltpu.async_copy(table_hbm.at[idx_v.at[pl.ds(off, REM)]],
                         rows0.at[pl.ds(0, REM)], sem0).wait()
        pltpu.sync_copy(rows0.at[pl.ds(0, REM)],
                        out_hbm.at[pl.ds(base + off, REM)])

    return k


def _make_sc_gather2(d):
    @functools.partial(
        pl.kernel,
        out_type=jax.ShapeDtypeStruct((N_EDGES, d), jnp.float32),
        mesh=_sc_mesh(),
        scratch_types=[pltpu.VMEM((EPW,), jnp.int32),
                       pltpu.VMEM((EPW,), jnp.int32),
                       pltpu.VMEM((CH, d), jnp.float32),
                       pltpu.VMEM((CH, d), jnp.float32),
                       pltpu.SemaphoreType.DMA,
                       pltpu.SemaphoreType.DMA],
    )
    def k(ta_hbm, tb_hbm, ia_hbm, ib_hbm, out_hbm,
          ia_v, ib_v, rows0, rows1, sem0, sem1):
        base = _wid() * EPW
        pltpu.sync_copy(ia_hbm.at[pl.ds(base, EPW)], ia_v)
        pltpu.sync_copy(ib_hbm.at[pl.ds(base, EPW)], ib_v)

        def gath(j, buf, sem):
            pltpu.async_copy(
                ta_hbm.at[ia_v.at[pl.ds(j * CH, CH)]], buf, sem)

        def fin(j, buf, sem):
            pltpu.make_async_copy(ta_hbm.at[pl.ds(0, CH)], buf, sem).wait()
            pltpu.sync_copy(tb_hbm.at[ib_v.at[pl.ds(j * CH, CH)]], buf,
                            add=True)
            pltpu.sync_copy(buf, out_hbm.at[pl.ds(base + j * CH, CH)])

        gath(0, rows0, sem0)

        def body(t, _):
            j = t * 2
            gath(j + 1, rows1, sem1)
            fin(j, rows0, sem0)
            gath(j + 2, rows0, sem0)
            fin(j + 1, rows1, sem1)
            return _

        lax.fori_loop(0, (NCH - 1) // 2, body, 0)
        fin(NCH - 1, rows0, sem0)
        off = NCH * CH
        pltpu.async_copy(ta_hbm.at[ia_v.at[pl.ds(off, REM)]],
                         rows0.at[pl.ds(0, REM)], sem0).wait()
        pltpu.sync_copy(tb_hbm.at[ib_v.at[pl.ds(off, REM)]],
                        rows0.at[pl.ds(0, REM)], add=True)
        pltpu.sync_copy(rows0.at[pl.ds(0, REM)],
                        out_hbm.at[pl.ds(base + off, REM)])

    return k


def _make_sc_segsum():
    @functools.partial(
        pl.kernel,
        out_type=jax.ShapeDtypeStruct((SC_NC, NPAD, C), jnp.float32),
        mesh=_sc_mesh(),
        scratch_types=[pltpu.VMEM((NCH + 1, CH), jnp.int32),
                       pltpu.VMEM((CH, C), jnp.float32),
                       pltpu.VMEM((CH, C), jnp.float32),
                       pltpu.VMEM_SHARED((NPAD, C), jnp.float32),
                       pltpu.SemaphoreType.DMA,
                       pltpu.SemaphoreType.DMA],
    )
    def k(m_hbm, dst2d_hbm, out_hbm, idx_v, rows0, rows1, acc_sh,
          sem0, sem1):
        cid = lax.axis_index("c")
        sid = lax.axis_index("s")
        wid = _wid()
        base = wid * EPW

        def zrow(i, _):
            for t in range(C // 16):
                rows0[i, pl.ds(t * 16, 16)] = jnp.zeros((16,), jnp.float32)
            return _

        lax.fori_loop(0, CH, zrow, 0)
        row0 = sid * NPA
        for t in range(NPA // CH):
            pltpu.sync_copy(rows0, acc_sh.at[pl.ds(row0 + t * CH, CH)])
        pltpu.sync_copy(dst2d_hbm.at[pl.ds(wid * (NCH + 1), NCH + 1)], idx_v)
        plsc.subcore_barrier()

        def load(j, buf, sem):
            pltpu.async_copy(m_hbm.at[pl.ds(base + j * CH, CH)], buf, sem)

        def fin(j, buf, sem):
            pltpu.make_async_copy(m_hbm.at[pl.ds(0, CH)], buf, sem).wait()
            pltpu.sync_copy(buf, acc_sh.at[idx_v.at[j]], add=True)

        load(0, rows0, sem0)

        def body(t, _):
            j = t * 2
            load(j + 1, rows1, sem1)
            fin(j, rows0, sem0)
            load(j + 2, rows0, sem0)
            fin(j + 1, rows1, sem1)
            return _

        lax.fori_loop(0, (NCH - 1) // 2, body, 0)
        fin(NCH - 1, rows0, sem0)
        pltpu.sync_copy(m_hbm.at[pl.ds(base + NCH * CH, REM)],
                        rows1.at[pl.ds(0, REM)])
        pltpu.sync_copy(rows1, acc_sh.at[idx_v.at[NCH]], add=True)
        plsc.subcore_barrier()

        pltpu.sync_copy(acc_sh.at[pl.ds(row0, NPA)],
                        out_hbm.at[cid].at[pl.ds(row0, NPA)])

    return k


def _make_sc_count():
    @functools.partial(
        pl.kernel,
        out_type=jax.ShapeDtypeStruct((SC_NC, NPAD, C), jnp.float32),
        mesh=_sc_mesh(),
        scratch_types=[pltpu.VMEM((CH,), jnp.int32),
                       pltpu.VMEM((REM,), jnp.int32),
                       pltpu.VMEM((CH, C), jnp.float32),
                       pltpu.VMEM((CH, C), jnp.float32),
                       pltpu.VMEM_SHARED((NPAD, C), jnp.float32),
                       pltpu.SemaphoreType.DMA],
    )
    def k(dst_hbm, out_hbm, idx_v, idx8_v, ones_v, zc_v, acc_sh, sem1):
        cid = lax.axis_index("c")
        sid = lax.axis_index("s")
        base = _wid() * EPW

        def orow(i, _):
            for t in range(C // 16):
                ones_v[i, pl.ds(t * 16, 16)] = jnp.ones((16,), jnp.float32)
                zc_v[i, pl.ds(t * 16, 16)] = jnp.zeros((16,), jnp.float32)
            return _

        lax.fori_loop(0, CH, orow, 0)
        row0 = sid * NPA
        for t in range(NPA // CH):
            pltpu.sync_copy(zc_v, acc_sh.at[pl.ds(row0 + t * CH, CH)])
        plsc.subcore_barrier()

        def body(j, _):
            off = base + j * CH
            pltpu.async_copy(dst_hbm.at[pl.ds(off, CH)], idx_v, sem1).wait()
            pltpu.sync_copy(ones_v, acc_sh.at[idx_v], add=True)
            return _

        lax.fori_loop(0, NCH, body, 0)
        off = base + NCH * CH
        pltpu.sync_copy(dst_hbm.at[pl.ds(off, REM)], idx8_v)
        pltpu.sync_copy(ones_v.at[pl.ds(0, REM)], acc_sh.at[idx8_v], add=True)
        plsc.subcore_barrier()

        pltpu.sync_copy(acc_sh.at[pl.ds(row0, NPA)],
                        out_hbm.at[cid].at[pl.ds(row0, NPA)])

    return k


def _bdot(a, w):
    return jnp.dot(a.astype(jnp.bfloat16), w.astype(jnp.bfloat16),
                   preferred_element_type=jnp.float32)


def _full(shape):
    return pl.BlockSpec(shape, lambda i: (0,) * len(shape))


def _rows(bs, width):
    return pl.BlockSpec((bs, width), lambda i: (i, 0))


def _node_mm_body(x_ref, w_ref, o_ref):
    o_ref[...] = jnp.dot(x_ref[...], w_ref[...],
                         preferred_element_type=jnp.float32)


def _node_mm(x, w):
    n, k = x.shape
    m = w.shape[1]
    return pl.pallas_call(
        _node_mm_body,
        grid=(n // BN,),
        in_specs=[_rows(BN, k), _full((k, m))],
        out_specs=_rows(BN, m),
        out_shape=jax.ShapeDtypeStruct((n, m), jnp.float32),
    )(x, w)


def _nc_tail_body(h_ref, w_ref, b_ref, o_ref):
    z = jnp.maximum(h_ref[...] + b_ref[0], 0.0)
    for l in range(3):
        z = jnp.maximum(_bdot(z, w_ref[l]) + b_ref[l + 1], 0.0)
    o_ref[...] = _bdot(z, w_ref[3]) + b_ref[4]


def _nc_tail(h, w_stack, b_stack):
    e = h.shape[0]
    return pl.pallas_call(
        _nc_tail_body,
        grid=(e // BE,),
        in_specs=[_rows(BE, C), _full((4, C, C)), _full((5, 1, C))],
        out_specs=_rows(BE, C),
        out_shape=jax.ShapeDtypeStruct((e, C), jnp.float32),
    )(h, w_stack, b_stack)


def _node_upd_body(x_ref, s_ref, c_ref, o_ref):
    s = s_ref[0] + s_ref[1]
    cnt = jnp.maximum((c_ref[0] + c_ref[1])[:, :1], 1.0)
    o_ref[...] = jnp.maximum(x_ref[...] + s / cnt, 0.0)


def _node_update(x, s_part, cnt_part):
    return pl.pallas_call(
        _node_upd_body,
        grid=(N_NODES // BN,),
        in_specs=[_rows(BN, C),
                  pl.BlockSpec((2, BN, C), lambda i: (0, i, 0)),
                  pl.BlockSpec((2, BN, C), lambda i: (0, i, 0))],
        out_specs=_rows(BN, C),
        out_shape=jax.ShapeDtypeStruct((N_NODES, C), jnp.float32),
    )(x, s_part, cnt_part)


def _mid_body(xs_ref, xd_ref, ang_ref,
              ew_ref, eb_ref, wfe_ref, wfa_ref, bf_ref,
              vw1_ref, vw_ref, vb_ref,
              e1_ref, m2_ref, sl_ref):
    f32 = jnp.float32
    xs = xs_ref[...]
    xd = xd_ref[...]
    w1a = ew_ref[0, :C]
    w1b = ew_ref[0, C:]
    hij = jnp.maximum(_bdot(xs, w1a) + _bdot(xd, w1b) + eb_ref[0], 0.0)
    hji = jnp.maximum(_bdot(xd, w1a) + _bdot(xs, w1b) + eb_ref[0], 0.0)
    for l in range(1, 4):
        w = ew_ref[l, :2 * C]
        hij = jnp.maximum(_bdot(hij, w) + eb_ref[l], 0.0)
        hji = jnp.maximum(_bdot(hji, w) + eb_ref[l], 0.0)
    w = ew_ref[4, :2 * C]
    fij = _bdot(hij, w) + eb_ref[4]
    fji = _bdot(hji, w) + eb_ref[4]
    d = fij - fji

    @pl.when(pl.program_id(0) == 0)
    def _():
        sl_ref[...] = jnp.zeros((1, 1), jnp.float32)

    sl_ref[...] += jnp.sum(d * d).reshape(1, 1)
    e = 0.5 * (fij + fji)
    e1_ref[...] = jnp.maximum(
        jnp.dot(e, wfe_ref[...], preferred_element_type=f32)
        + ang_ref[...] * wfa_ref[...] + bf_ref[...], 0.0)
    v1a = vw1_ref[:C]
    v1b = vw1_ref[C:]
    g = jnp.maximum(_bdot(xs, v1a) + _bdot(xd, v1b) + vb_ref[0], 0.0)
    for l in range(3):
        g = jnp.maximum(_bdot(g, vw_ref[l]) + vb_ref[l + 1], 0.0)
    m2_ref[...] = _bdot(g, vw_ref[3]) + vb_ref[4]


def _mid(xs, xd, ang, ew, eb, wfe, wfa, bf, vw1, vw, vb):
    e = xs.shape[0]
    return pl.pallas_call(
        _mid_body,
        grid=(e // BE,),
        in_specs=[_rows(BE, C), _rows(BE, C), _rows(BE, 1),
                  _full((5, 2 * C, 2 * C)), _full((5, 1, 2 * C)),
                  _full((2 * C, 2 * C)), _full((1, 2 * C)), _full((1, 2 * C)),
                  _full((2 * C, C)), _full((4, C, C)), _full((5, 1, C))],
        out_specs=[_rows(BE, 2 * C), _rows(BE, C),
                   pl.BlockSpec((1, 1), lambda i: (0, 0))],
        out_shape=[jax.ShapeDtypeStruct((e, 2 * C), jnp.float32),
                   jax.ShapeDtypeStruct((e, C), jnp.float32),
                   jax.ShapeDtypeStruct((1, 1), jnp.float32)],
    )(xs, xd, ang, ew, eb, wfe, wfa, bf, vw1, vw, vb)


def _ec2_body(xs_ref, xd_ref, e1_ref,
              uw1_ref, uw_ref, ub_ref, wfe_ref, wfi_ref, bf_ref,
              e2_ref, sl_ref):
    f32 = jnp.float32
    xs = xs_ref[...]
    xd = xd_ref[...]
    u1a = uw1_ref[:C]
    u1b = uw1_ref[C:]
    hij = jnp.maximum(_bdot(xs, u1a) + _bdot(xd, u1b) + ub_ref[0], 0.0)
    hji = jnp.maximum(_bdot(xd, u1a) + _bdot(xs, u1b) + ub_ref[0], 0.0)
    for l in range(3):
        w = uw_ref[l]
        hij = jnp.maximum(_bdot(hij, w) + ub_ref[l + 1], 0.0)
        hji = jnp.maximum(_bdot(hji, w) + ub_ref[l + 1], 0.0)
    w = uw_ref[3]
    fij = _bdot(hij, w) + ub_ref[4]
    fji = _bdot(hji, w) + ub_ref[4]
    d = fij - fji

    @pl.when(pl.program_id(0) == 0)
    def _():
        sl_ref[...] = jnp.zeros((1, 1), jnp.float32)

    sl_ref[...] += jnp.sum(d * d).reshape(1, 1)
    e = 0.5 * (fij + fji)
    e2_ref[...] = jnp.maximum(
        jnp.dot(e, wfe_ref[...], preferred_element_type=f32)
        + jnp.dot(e1_ref[...], wfi_ref[...], preferred_element_type=f32)
        + bf_ref[...], 0.0)


def _ec2(xs, xd, e1, uw1, uw, ub, wfe, wfi, bf):
    e = xs.shape[0]
    return pl.pallas_call(
        _ec2_body,
        grid=(e // BE,),
        in_specs=[_rows(BE, C), _rows(BE, C), _rows(BE, 2 * C),
                  _full((2 * C, C)), _full((4, C, C)), _full((5, 1, C)),
                  _full((C, C)), _full((2 * C, C)), _full((1, C))],
        out_specs=[_rows(BE, C), pl.BlockSpec((1, 1), lambda i: (0, 0))],
        out_shape=[jax.ShapeDtypeStruct((e, C), jnp.float32),
                   jax.ShapeDtypeStruct((1, 1), jnp.float32)],
    )(xs, xd, e1, uw1, uw, ub, wfe, wfi, bf)


def _stack_mlp(layers):
    w1 = layers[0][0]
    wt = jnp.stack([w for (w, _) in layers[1:]])
    bs = jnp.stack([b[None, :] for (_, b) in layers])
    return w1, wt, bs


def kernel(node_features, edge_index, angles, gt_edges,
           nc1, ec1_mlp, ec1_fuse, nc2, ec2_mlp, ec2_fuse):
    src, dst = edge_index[0], edge_index[1]
    x0 = node_features
    dstw = dst.reshape(SC_NW, EPW)
    pad = jnp.full((SC_NW, (NCH + 1) * CH - EPW), NPAD - 1, jnp.int32)
    dst2d = jnp.concatenate([dstw, pad], axis=1).reshape(-1, CH)
    gather = _make_sc_gather(C)
    gather2 = _make_sc_gather2(C)
    seg = _make_sc_segsum()
    cntp = _make_sc_count()(dst)

    n1w1, n1wt, n1b = _stack_mlp(nc1)
    ab1 = _node_mm(x0, jnp.concatenate([n1w1[:C], n1w1[C:]], axis=1))
    h1 = gather2(ab1[:, :C], ab1[:, C:], src, dst)
    m1 = _nc_tail(h1, n1wt, n1b)
    s1p = seg(m1, dst2d)
    x1 = _node_update(x0, s1p, cntp)

    xs1 = gather(x1, src)
    xd1 = gather(x1, dst)
    ew = jnp.stack([w for (w, _) in ec1_mlp])
    eb = jnp.stack([b[None, :] for (_, b) in ec1_mlp])
    wf1, bf1 = ec1_fuse
    n2w1, n2wt, n2b = _stack_mlp(nc2)
    e1, m2, sl1s = _mid(xs1, xd1, angles, ew, eb,
                        wf1[:2 * C], wf1[2 * C:2 * C + 1], bf1[None, :],
                        n2w1, n2wt, n2b)
    s2p = seg(m2, dst2d)
    x2 = _node_update(x1, s2p, cntp)

    xs2 = gather(x2, src)
    xd2 = gather(x2, dst)
    e2w1, e2wt, e2b = _stack_mlp(ec2_mlp)
    wf2, bf2 = ec2_fuse
    e2, sl2s = _ec2(xs2, xd2, e1, e2w1, e2wt, e2b,
                    wf2[:C], wf2[C:], bf2[None, :])

    sl1 = sl1s[0, 0] / (N_EDGES * 2 * C)
    sl2 = sl2s[0, 0] / (N_EDGES * C)
    return (e2, (sl1 + sl2) / 2.0)


# speedup vs baseline: 1.0005x; 1.0005x over previous
import functools

import jax
import jax.numpy as jnp
from jax import lax
from jax.experimental import pallas as pl
from jax.experimental.pallas import tpu as pltpu
from jax.experimental.pallas import tpu_sc as plsc

N_NODES = 10000
N_EDGES = 160000
C = 128
BE = 2000
BN = 2000

SC_NC = 2
SC_NS = 16
SC_NW = SC_NC * SC_NS
EPW = N_EDGES // SC_NW
CH = 128
NCH = EPW // CH
REM = EPW - NCH * CH
NPAD = 10240
NPA = NPAD // SC_NS


def _sc_mesh():
    return plsc.VectorSubcoreMesh(core_axis_name="c", subcore_axis_name="s",
                                  num_cores=SC_NC, num_subcores=SC_NS)


def _wid():
    return lax.axis_index("s") * SC_NC + lax.axis_index("c")


def _make_sc_gather(d, dt=jnp.float32):
    @functools.partial(
        pl.kernel,
        out_type=jax.ShapeDtypeStruct((N_EDGES, d), dt),
        mesh=_sc_mesh(),
        scratch_types=[pltpu.VMEM((EPW,), jnp.int32),
                       pltpu.VMEM((CH, d), dt),
                       pltpu.VMEM((CH, d), dt),
                       pltpu.SemaphoreType.DMA,
                       pltpu.SemaphoreType.DMA],
    )
    def k(table_hbm, idx_hbm, out_hbm, idx_v, rows0, rows1, sem0, sem1):
        base = _wid() * EPW
        pltpu.sync_copy(idx_hbm.at[pl.ds(base, EPW)], idx_v)

        def gath(j, buf, sem):
            return pltpu.async_copy(
                table_hbm.at[idx_v.at[pl.ds(j * CH, CH)]], buf, sem)

        def put(j, buf):
            pltpu.sync_copy(buf, out_hbm.at[pl.ds(base + j * CH, CH)])

        def drain(buf, sem):
            pltpu.make_async_copy(table_hbm.at[pl.ds(0, CH)], buf, sem).wait()

        gath(0, rows0, sem0)

        def body(t, _):
            j = t * 2
            gath(j + 1, rows1, sem1)
            drain(rows0, sem0)
            put(j, rows0)
            gath(j + 2, rows0, sem0)
            drain(rows1, sem1)
            put(j + 1, rows1)
            return _

        lax.fori_loop(0, (NCH - 1) // 2, body, 0)
        drain(rows0, sem0)
        put(NCH - 1, rows0)
        off = NCH * CH
        pltpu.async_copy(table_hbm.at[idx_v.at[pl.ds(off, REM)]],
                         rows0.at[pl.ds(0, REM)], sem0).wait()
        pltpu.sync_copy(rows0.at[pl.ds(0, REM)],
                        out_hbm.at[pl.ds(base + off, REM)])

    return k


def _make_sc_gather2(d):
    @functools.partial(
        pl.kernel,
        out_type=jax.ShapeDtypeStruct((N_EDGES, d), jnp.float32),
        mesh=_sc_mesh(),
        scratch_types=[pltpu.VMEM((EPW,), jnp.int32),
                       pltpu.VMEM((EPW,), jnp.int32),
                       pltpu.VMEM((CH, d), jnp.float32),
                       pltpu.VMEM((CH, d), jnp.float32),
                       pltpu.SemaphoreType.DMA,
                       pltpu.SemaphoreType.DMA],
    )
    def k(ta_hbm, tb_hbm, ia_hbm, ib_hbm, out_hbm,
          ia_v, ib_v, rows0, rows1, sem0, sem1):
        base = _wid() * EPW
        pltpu.sync_copy(ia_hbm.at[pl.ds(base, EPW)], ia_v)
        pltpu.sync_copy(ib_hbm.at[pl.ds(base, EPW)], ib_v)

        def gath(j, buf, sem):
            pltpu.async_copy(
                ta_hbm.at[ia_v.at[pl.ds(j * CH, CH)]], buf, sem)

        def fin(j, buf, sem):
            pltpu.make_async_copy(ta_hbm.at[pl.ds(0, CH)], buf, sem).wait()
            pltpu.sync_copy(tb_hbm.at[ib_v.at[pl.ds(j * CH, CH)]], buf,
                            add=True)
            pltpu.sync_copy(buf, out_hbm.at[pl.ds(base + j * CH, CH)])

        gath(0, rows0, sem0)

        def body(t, _):
            j = t * 2
            gath(j + 1, rows1, sem1)
            fin(j, rows0, sem0)
            gath(j + 2, rows0, sem0)
            fin(j + 1, rows1, sem1)
            return _

        lax.fori_loop(0, (NCH - 1) // 2, body, 0)
        fin(NCH - 1, rows0, sem0)
        off = NCH * CH
        pltpu.async_copy(ta_hbm.at[ia_v.at[pl.ds(off, REM)]],
                         rows0.at[pl.ds(0, REM)], sem0).wait()
        pltpu.sync_copy(tb_hbm.at[ib_v.at[pl.ds(off, REM)]],
                        rows0.at[pl.ds(0, REM)], add=True)
        pltpu.sync_copy(rows0.at[pl.ds(0, REM)],
                        out_hbm.at[pl.ds(base + off, REM)])

    return k


def _make_sc_segsum():
    @functools.partial(
        pl.kernel,
        out_type=jax.ShapeDtypeStruct((SC_NC, NPAD, C), jnp.float32),
        mesh=_sc_mesh(),
        scratch_types=[pltpu.VMEM((NCH + 1, CH), jnp.int32),
                       pltpu.VMEM((CH, C), jnp.float32),
                       pltpu.VMEM((CH, C), jnp.float32),
                       pltpu.VMEM_SHARED((NPAD, C), jnp.float32),
                       pltpu.SemaphoreType.DMA,
                       pltpu.SemaphoreType.DMA],
    )
    def k(m_hbm, dst2d_hbm, out_hbm, idx_v, rows0, rows1, acc_sh,
          sem0, sem1):
        cid = lax.axis_index("c")
        sid = lax.axis_index("s")
        wid = _wid()
        base = wid * EPW

        def zrow(i, _):
            for t in range(C // 16):
                rows0[i, pl.ds(t * 16, 16)] = jnp.zeros((16,), jnp.float32)
            return _

        lax.fori_loop(0, CH, zrow, 0)
        row0 = sid * NPA
        for t in range(NPA // CH):
            pltpu.sync_copy(rows0, acc_sh.at[pl.ds(row0 + t * CH, CH)])
        pltpu.sync_copy(dst2d_hbm.at[pl.ds(wid * (NCH + 1), NCH + 1)], idx_v)
        plsc.subcore_barrier()

        def load(j, buf, sem):
            pltpu.async_copy(m_hbm.at[pl.ds(base + j * CH, CH)], buf, sem)

        def fin(j, buf, sem):
            pltpu.make_async_copy(m_hbm.at[pl.ds(0, CH)], buf, sem).wait()
            pltpu.sync_copy(buf, acc_sh.at[idx_v.at[j]], add=True)

        load(0, rows0, sem0)

        def body(t, _):
            j = t * 2
            load(j + 1, rows1, sem1)
            fin(j, rows0, sem0)
            load(j + 2, rows0, sem0)
            fin(j + 1, rows1, sem1)
            return _

        lax.fori_loop(0, (NCH - 1) // 2, body, 0)
        fin(NCH - 1, rows0, sem0)
        pltpu.sync_copy(m_hbm.at[pl.ds(base + NCH * CH, REM)],
                        rows1.at[pl.ds(0, REM)])
        pltpu.sync_copy(rows1, acc_sh.at[idx_v.at[NCH]], add=True)
        plsc.subcore_barrier()

        pltpu.sync_copy(acc_sh.at[pl.ds(row0, NPA)],
                        out_hbm.at[cid].at[pl.ds(row0, NPA)])

    return k


def _make_sc_count():
    @functools.partial(
        pl.kernel,
        out_type=jax.ShapeDtypeStruct((SC_NC, NPAD, C), jnp.float32),
        mesh=_sc_mesh(),
        scratch_types=[pltpu.VMEM((CH,), jnp.int32),
                       pltpu.VMEM((REM,), jnp.int32),
                       pltpu.VMEM((CH, C), jnp.float32),
                       pltpu.VMEM((CH, C), jnp.float32),
                       pltpu.VMEM_SHARED((NPAD, C), jnp.float32),
                       pltpu.SemaphoreType.DMA],
    )
    def k(dst_hbm, out_hbm, idx_v, idx8_v, ones_v, zc_v, acc_sh, sem1):
        cid = lax.axis_index("c")
        sid = lax.axis_index("s")
        base = _wid() * EPW

        def orow(i, _):
            for t in range(C // 16):
                ones_v[i, pl.ds(t * 16, 16)] = jnp.ones((16,), jnp.float32)
                zc_v[i, pl.ds(t * 16, 16)] = jnp.zeros((16,), jnp.float32)
            return _

        lax.fori_loop(0, CH, orow, 0)
        row0 = sid * NPA
        for t in range(NPA // CH):
            pltpu.sync_copy(zc_v, acc_sh.at[pl.ds(row0 + t * CH, CH)])
        plsc.subcore_barrier()

        def body(j, _):
            off = base + j * CH
            pltpu.async_copy(dst_hbm.at[pl.ds(off, CH)], idx_v, sem1).wait()
            pltpu.sync_copy(ones_v, acc_sh.at[idx_v], add=True)
            return _

        lax.fori_loop(0, NCH, body, 0)
        off = base + NCH * CH
        pltpu.sync_copy(dst_hbm.at[pl.ds(off, REM)], idx8_v)
        pltpu.sync_copy(ones_v.at[pl.ds(0, REM)], acc_sh.at[idx8_v], add=True)
        plsc.subcore_barrier()

        pltpu.sync_copy(acc_sh.at[pl.ds(row0, NPA)],
                        out_hbm.at[cid].at[pl.ds(row0, NPA)])

    return k


def _bdot(a, w):
    return jnp.dot(a.astype(jnp.bfloat16), w.astype(jnp.bfloat16),
                   preferred_element_type=jnp.float32)


def _full(shape):
    return pl.BlockSpec(shape, lambda i: (0,) * len(shape))


def _rows(bs, width):
    return pl.BlockSpec((bs, width), lambda i: (i, 0))


def _node_mm_body(x_ref, w_ref, o_ref):
    o_ref[...] = jnp.dot(x_ref[...], w_ref[...],
                         preferred_element_type=jnp.float32)


def _node_mm(x, w):
    n, k = x.shape
    m = w.shape[1]
    return pl.pallas_call(
        _node_mm_body,
        grid=(n // BN,),
        in_specs=[_rows(BN, k), _full((k, m))],
        out_specs=_rows(BN, m),
        out_shape=jax.ShapeDtypeStruct((n, m), jnp.float32),
    )(x, w)


def _nc_tail_body(h_ref, w_ref, b_ref, o_ref):
    z = jnp.maximum(h_ref[...] + b_ref[0], 0.0)
    for l in range(3):
        z = jnp.maximum(_bdot(z, w_ref[l]) + b_ref[l + 1], 0.0)
    o_ref[...] = _bdot(z, w_ref[3]) + b_ref[4]


def _nc_tail(h, w_stack, b_stack):
    e = h.shape[0]
    return pl.pallas_call(
        _nc_tail_body,
        grid=(e // BE,),
        in_specs=[_rows(BE, C), _full((4, C, C)), _full((5, 1, C))],
        out_specs=_rows(BE, C),
        out_shape=jax.ShapeDtypeStruct((e, C), jnp.float32),
    )(h, w_stack, b_stack)


def _node_upd_body(x_ref, s_ref, c_ref, o_ref):
    s = s_ref[0] + s_ref[1]
    cnt = jnp.maximum((c_ref[0] + c_ref[1])[:, :1], 1.0)
    o_ref[...] = jnp.maximum(x_ref[...] + s / cnt, 0.0)


def _node_update(x, s_part, cnt_part):
    return pl.pallas_call(
        _node_upd_body,
        grid=(N_NODES // BN,),
        in_specs=[_rows(BN, C),
                  pl.BlockSpec((2, BN, C), lambda i: (0, i, 0)),
                  pl.BlockSpec((2, BN, C), lambda i: (0, i, 0))],
        out_specs=_rows(BN, C),
        out_shape=jax.ShapeDtypeStruct((N_NODES, C), jnp.float32),
    )(x, s_part, cnt_part)


def _mid_body(xs_ref, xd_ref, ang_ref,
              ew_ref, eb_ref, wfe_ref, wfa_ref, bf_ref,
              vw1_ref, vw_ref, vb_ref,
              e1_ref, m2_ref, sl_ref):
    f32 = jnp.float32
    xs = xs_ref[...]
    xd = xd_ref[...]
    w1a = ew_ref[0, :C]
    w1b = ew_ref[0, C:]
    hij = jnp.maximum(_bdot(xs, w1a) + _bdot(xd, w1b) + eb_ref[0], 0.0)
    hji = jnp.maximum(_bdot(xd, w1a) + _bdot(xs, w1b) + eb_ref[0], 0.0)
    for l in range(1, 4):
        w = ew_ref[l, :2 * C]
        hij = jnp.maximum(_bdot(hij, w) + eb_ref[l], 0.0)
        hji = jnp.maximum(_bdot(hji, w) + eb_ref[l], 0.0)
    w = ew_ref[4, :2 * C]
    fij = _bdot(hij, w) + eb_ref[4]
    fji = _bdot(hji, w) + eb_ref[4]
    d = fij - fji

    @pl.when(pl.program_id(0) == 0)
    def _():
        sl_ref[...] = jnp.zeros((1, 1), jnp.float32)

    sl_ref[...] += jnp.sum(d * d).reshape(1, 1)
    e = 0.5 * (fij + fji)
    e1_ref[...] = jnp.maximum(
        jnp.dot(e, wfe_ref[...], preferred_element_type=f32)
        + ang_ref[...] * wfa_ref[...] + bf_ref[...], 0.0)
    v1a = vw1_ref[:C]
    v1b = vw1_ref[C:]
    g = jnp.maximum(_bdot(xs, v1a) + _bdot(xd, v1b) + vb_ref[0], 0.0)
    for l in range(3):
        g = jnp.maximum(_bdot(g, vw_ref[l]) + vb_ref[l + 1], 0.0)
    m2_ref[...] = _bdot(g, vw_ref[3]) + vb_ref[4]


def _mid(xs, xd, ang, ew, eb, wfe, wfa, bf, vw1, vw, vb):
    e = xs.shape[0]
    return pl.pallas_call(
        _mid_body,
        grid=(e // BE,),
        in_specs=[_rows(BE, C), _rows(BE, C), _rows(BE, 1),
                  _full((5, 2 * C, 2 * C)), _full((5, 1, 2 * C)),
                  _full((2 * C, 2 * C)), _full((1, 2 * C)), _full((1, 2 * C)),
                  _full((2 * C, C)), _full((4, C, C)), _full((5, 1, C))],
        out_specs=[_rows(BE, 2 * C), _rows(BE, C),
                   pl.BlockSpec((1, 1), lambda i: (0, 0))],
        out_shape=[jax.ShapeDtypeStruct((e, 2 * C), jnp.float32),
                   jax.ShapeDtypeStruct((e, C), jnp.float32),
                   jax.ShapeDtypeStruct((1, 1), jnp.float32)],
    )(xs, xd, ang, ew, eb, wfe, wfa, bf, vw1, vw, vb)


def _ec2_body(xs_ref, xd_ref, e1_ref,
              uw1_ref, uw_ref, ub_ref, wfe_ref, wfi_ref, bf_ref,
              e2_ref, sl_ref):
    f32 = jnp.float32
    xs = xs_ref[...]
    xd = xd_ref[...]
    u1a = uw1_ref[:C]
    u1b = uw1_ref[C:]
    hij = jnp.maximum(_bdot(xs, u1a) + _bdot(xd, u1b) + ub_ref[0], 0.0)
    hji = jnp.maximum(_bdot(xd, u1a) + _bdot(xs, u1b) + ub_ref[0], 0.0)
    for l in range(3):
        w = uw_ref[l]
        hij = jnp.maximum(_bdot(hij, w) + ub_ref[l + 1], 0.0)
        hji = jnp.maximum(_bdot(hji, w) + ub_ref[l + 1], 0.0)
    w = uw_ref[3]
    fij = _bdot(hij, w) + ub_ref[4]
    fji = _bdot(hji, w) + ub_ref[4]
    d = fij - fji

    @pl.when(pl.program_id(0) == 0)
    def _():
        sl_ref[...] = jnp.zeros((1, 1), jnp.float32)

    sl_ref[...] += jnp.sum(d * d).reshape(1, 1)
    e = 0.5 * (fij + fji)
    e2_ref[...] = jnp.maximum(
        jnp.dot(e, wfe_ref[...], preferred_element_type=f32)
        + jnp.dot(e1_ref[...], wfi_ref[...], preferred_element_type=f32)
        + bf_ref[...], 0.0)


def _ec2(xs, xd, e1, uw1, uw, ub, wfe, wfi, bf):
    e = xs.shape[0]
    return pl.pallas_call(
        _ec2_body,
        grid=(e // BE,),
        in_specs=[_rows(BE, C), _rows(BE, C), _rows(BE, 2 * C),
                  _full((2 * C, C)), _full((4, C, C)), _full((5, 1, C)),
                  _full((C, C)), _full((2 * C, C)), _full((1, C))],
        out_specs=[_rows(BE, C), pl.BlockSpec((1, 1), lambda i: (0, 0))],
        out_shape=[jax.ShapeDtypeStruct((e, C), jnp.float32),
                   jax.ShapeDtypeStruct((1, 1), jnp.float32)],
    )(xs, xd, e1, uw1, uw, ub, wfe, wfi, bf)


def _stack_mlp(layers):
    w1 = layers[0][0]
    wt = jnp.stack([w for (w, _) in layers[1:]])
    bs = jnp.stack([b[None, :] for (_, b) in layers])
    return w1, wt, bs


def kernel(node_features, edge_index, angles, gt_edges,
           nc1, ec1_mlp, ec1_fuse, nc2, ec2_mlp, ec2_fuse):
    src, dst = edge_index[0], edge_index[1]
    x0 = node_features
    dstw = dst.reshape(SC_NW, EPW)
    pad = jnp.full((SC_NW, (NCH + 1) * CH - EPW), NPAD - 1, jnp.int32)
    dst2d = jnp.concatenate([dstw, pad], axis=1).reshape(-1, CH)
    gather = _make_sc_gather(C)
    gather2 = _make_sc_gather2(C)
    seg = _make_sc_segsum()
    cntp = _make_sc_count()(dst)

    n1w1, n1wt, n1b = _stack_mlp(nc1)
    ab1 = _node_mm(x0, jnp.concatenate([n1w1[:C], n1w1[C:]], axis=1))
    h1 = gather2(ab1[:, :C], ab1[:, C:], src, dst)
    m1 = _nc_tail(h1, n1wt.astype(jnp.bfloat16), n1b)
    s1p = seg(m1, dst2d)
    x1 = _node_update(x0, s1p, cntp)

    xs1 = gather(x1, src)
    xd1 = gather(x1, dst)
    ew = jnp.stack([w for (w, _) in ec1_mlp])
    eb = jnp.stack([b[None, :] for (_, b) in ec1_mlp])
    wf1, bf1 = ec1_fuse
    n2w1, n2wt, n2b = _stack_mlp(nc2)
    e1, m2, sl1s = _mid(xs1, xd1, angles, ew.astype(jnp.bfloat16), eb,
                        wf1[:2 * C], wf1[2 * C:2 * C + 1], bf1[None, :],
                        n2w1.astype(jnp.bfloat16),
                        n2wt.astype(jnp.bfloat16), n2b)
    s2p = seg(m2, dst2d)
    x2 = _node_update(x1, s2p, cntp)

    xs2 = gather(x2, src)
    xd2 = gather(x2, dst)
    e2w1, e2wt, e2b = _stack_mlp(ec2_mlp)
    wf2, bf2 = ec2_fuse
    e2, sl2s = _ec2(xs2, xd2, e1, e2w1.astype(jnp.bfloat16),
                    e2wt.astype(jnp.bfloat16), e2b,
                    wf2[:C], wf2[C:], bf2[None, :])

    sl1 = sl1s[0, 0] / (N_EDGES * 2 * C)
    sl2 = sl2s[0, 0] / (N_EDGES * C)
    return (e2, (sl1 + sl2) / 2.0)


# speedup vs baseline: 1.1600x; 1.1594x over previous
import functools

import jax
import jax.numpy as jnp
from jax import lax
from jax.experimental import pallas as pl
from jax.experimental.pallas import tpu as pltpu
from jax.experimental.pallas import tpu_sc as plsc

N_NODES = 10000
N_EDGES = 160000
C = 128
BE = 1280
BN = 2000
E1 = 81920
E2 = N_EDGES - E1

SC_NC = 2
SC_NS = 16
SC_NW = SC_NC * SC_NS
CH = 128
NPAD = 10240
NPA = NPAD // SC_NS


def _geom(e):
    epw = e // SC_NW
    return epw, epw // CH, epw % CH


def _sc_mesh():
    return plsc.VectorSubcoreMesh(core_axis_name="c", subcore_axis_name="s",
                                  num_cores=SC_NC, num_subcores=SC_NS)


def _wid():
    return lax.axis_index("s") * SC_NC + lax.axis_index("c")


def _pipe(nch, gath, fin):
    gath(0, 0)

    def body(t, _):
        j = t * 2
        gath(j + 1, 1)
        fin(j, 0)
        gath(j + 2, 0)
        fin(j + 1, 1)
        return _

    lax.fori_loop(0, (nch - 1) // 2, body, 0)
    if nch % 2:
        fin(nch - 1, 0)
    else:
        gath(nch - 1, 1)
        fin(nch - 2, 0)
        fin(nch - 1, 1)


def _make_sc_gather(d, e):
    epw, nch, rem = _geom(e)

    @functools.partial(
        pl.kernel,
        out_type=jax.ShapeDtypeStruct((e, d), jnp.float32),
        mesh=_sc_mesh(),
        scratch_types=[pltpu.VMEM((epw,), jnp.int32),
                       pltpu.VMEM((CH, d), jnp.float32),
                       pltpu.VMEM((CH, d), jnp.float32),
                       pltpu.SemaphoreType.DMA,
                       pltpu.SemaphoreType.DMA],
    )
    def k(table_hbm, idx_hbm, out_hbm, idx_v, rows0, rows1, sem0, sem1):
        base = _wid() * epw
        pltpu.sync_copy(idx_hbm.at[pl.ds(base, epw)], idx_v)
        bufs = (rows0, rows1)
        sems = (sem0, sem1)

        def gath(j, p):
            pltpu.async_copy(
                table_hbm.at[idx_v.at[pl.ds(j * CH, CH)]], bufs[p], sems[p])

        def fin(j, p):
            pltpu.make_async_copy(
                table_hbm.at[pl.ds(0, CH)], bufs[p], sems[p]).wait()
            pltpu.sync_copy(bufs[p], out_hbm.at[pl.ds(base + j * CH, CH)])

        _pipe(nch, gath, fin)
        if rem:
            off = nch * CH
            pltpu.async_copy(table_hbm.at[idx_v.at[pl.ds(off, rem)]],
                             rows0.at[pl.ds(0, rem)], sem0).wait()
            pltpu.sync_copy(rows0.at[pl.ds(0, rem)],
                            out_hbm.at[pl.ds(base + off, rem)])

    return k


def _make_sc_gather2(d, e):
    epw, nch, rem = _geom(e)

    @functools.partial(
        pl.kernel,
        out_type=jax.ShapeDtypeStruct((e, d), jnp.float32),
        mesh=_sc_mesh(),
        scratch_types=[pltpu.VMEM((epw,), jnp.int32),
                       pltpu.VMEM((epw,), jnp.int32),
                       pltpu.VMEM((CH, d), jnp.float32),
                       pltpu.VMEM((CH, d), jnp.float32),
                       pltpu.SemaphoreType.DMA,
                       pltpu.SemaphoreType.DMA],
    )
    def k(ta_hbm, tb_hbm, ia_hbm, ib_hbm, out_hbm,
          ia_v, ib_v, rows0, rows1, sem0, sem1):
        base = _wid() * epw
        pltpu.sync_copy(ia_hbm.at[pl.ds(base, epw)], ia_v)
        pltpu.sync_copy(ib_hbm.at[pl.ds(base, epw)], ib_v)
        bufs = (rows0, rows1)
        sems = (sem0, sem1)

        def gath(j, p):
            pltpu.async_copy(
                ta_hbm.at[ia_v.at[pl.ds(j * CH, CH)]], bufs[p], sems[p])

        def fin(j, p):
            pltpu.make_async_copy(
                ta_hbm.at[pl.ds(0, CH)], bufs[p], sems[p]).wait()
            pltpu.sync_copy(tb_hbm.at[ib_v.at[pl.ds(j * CH, CH)]], bufs[p],
                            add=True)
            pltpu.sync_copy(bufs[p], out_hbm.at[pl.ds(base + j * CH, CH)])

        _pipe(nch, gath, fin)
        if rem:
            off = nch * CH
            pltpu.async_copy(ta_hbm.at[ia_v.at[pl.ds(off, rem)]],
                             rows0.at[pl.ds(0, rem)], sem0).wait()
            pltpu.sync_copy(tb_hbm.at[ib_v.at[pl.ds(off, rem)]],
                            rows0.at[pl.ds(0, rem)], add=True)
            pltpu.sync_copy(rows0.at[pl.ds(0, rem)],
                            out_hbm.at[pl.ds(base + off, rem)])

    return k


def _make_sc_segsum(e):
    epw, nch, rem = _geom(e)
    nrows = -(-(nch + (1 if rem else 0)) // 8) * 8

    @functools.partial(
        pl.kernel,
        out_type=jax.ShapeDtypeStruct((SC_NC, NPAD, C), jnp.float32),
        mesh=_sc_mesh(),
        scratch_types=[pltpu.VMEM((nrows, CH), jnp.int32),
                       pltpu.VMEM((CH, C), jnp.float32),
                       pltpu.VMEM((CH, C), jnp.float32),
                       pltpu.VMEM_SHARED((NPAD, C), jnp.float32),
                       pltpu.SemaphoreType.DMA,
                       pltpu.SemaphoreType.DMA],
    )
    def k(m_hbm, dst2d_hbm, out_hbm, idx_v, rows0, rows1, acc_sh,
          sem0, sem1):
        cid = lax.axis_index("c")
        sid = lax.axis_index("s")
        wid = _wid()
        base = wid * epw

        def zrow(i, _):
            for t in range(C // 16):
                rows0[i, pl.ds(t * 16, 16)] = jnp.zeros((16,), jnp.float32)
            return _

        lax.fori_loop(0, CH, zrow, 0)
        row0 = sid * NPA
        for t in range(NPA // CH):
            pltpu.sync_copy(rows0, acc_sh.at[pl.ds(row0 + t * CH, CH)])
        pltpu.sync_copy(dst2d_hbm.at[pl.ds(wid * nrows, nrows)], idx_v)
        plsc.subcore_barrier()

        bufs = (rows0, rows1)
        sems = (sem0, sem1)

        def load(j, p):
            pltpu.async_copy(m_hbm.at[pl.ds(base + j * CH, CH)], bufs[p],
                             sems[p])

        def fin(j, p):
            pltpu.make_async_copy(m_hbm.at[pl.ds(0, CH)], bufs[p],
                                  sems[p]).wait()
            pltpu.sync_copy(bufs[p], acc_sh.at[idx_v.at[j]], add=True)

        _pipe(nch, load, fin)
        if rem:
            pltpu.sync_copy(m_hbm.at[pl.ds(base + nch * CH, rem)],
                            rows1.at[pl.ds(0, rem)])
            pltpu.sync_copy(rows1, acc_sh.at[idx_v.at[nch]], add=True)
        plsc.subcore_barrier()

        pltpu.sync_copy(acc_sh.at[pl.ds(row0, NPA)],
                        out_hbm.at[cid].at[pl.ds(row0, NPA)])

    return k


def _make_sc_count():
    epw, nch, rem = _geom(N_EDGES)

    @functools.partial(
        pl.kernel,
        out_type=jax.ShapeDtypeStruct((SC_NC, NPAD, C), jnp.float32),
        mesh=_sc_mesh(),
        scratch_types=[pltpu.VMEM((CH,), jnp.int32),
                       pltpu.VMEM((rem,), jnp.int32),
                       pltpu.VMEM((CH, C), jnp.float32),
                       pltpu.VMEM((CH, C), jnp.float32),
                       pltpu.VMEM_SHARED((NPAD, C), jnp.float32),
                       pltpu.SemaphoreType.DMA],
    )
    def k(dst_hbm, out_hbm, idx_v, idx8_v, ones_v, zc_v, acc_sh, sem1):
        cid = lax.axis_index("c")
        sid = lax.axis_index("s")
        base = _wid() * epw

        def orow(i, _):
            for t in range(C // 16):
                ones_v[i, pl.ds(t * 16, 16)] = jnp.ones((16,), jnp.float32)
                zc_v[i, pl.ds(t * 16, 16)] = jnp.zeros((16,), jnp.float32)
            return _

        lax.fori_loop(0, CH, orow, 0)
        row0 = sid * NPA
        for t in range(NPA // CH):
            pltpu.sync_copy(zc_v, acc_sh.at[pl.ds(row0 + t * CH, CH)])
        plsc.subcore_barrier()

        def body(j, _):
            off = base + j * CH
            pltpu.async_copy(dst_hbm.at[pl.ds(off, CH)], idx_v, sem1).wait()
            pltpu.sync_copy(ones_v, acc_sh.at[idx_v], add=True)
            return _

        lax.fori_loop(0, nch, body, 0)
        off = base + nch * CH
        pltpu.sync_copy(dst_hbm.at[pl.ds(off, rem)], idx8_v)
        pltpu.sync_copy(ones_v.at[pl.ds(0, rem)], acc_sh.at[idx8_v],
                        add=True)
        plsc.subcore_barrier()

        pltpu.sync_copy(acc_sh.at[pl.ds(row0, NPA)],
                        out_hbm.at[cid].at[pl.ds(row0, NPA)])

    return k


def _full(shape):
    return pl.BlockSpec(shape, lambda i: (0,) * len(shape))


def _rows(bs, width):
    return pl.BlockSpec((bs, width), lambda i: (i, 0))


def _node_mm_body(x_ref, w_ref, o_ref):
    o_ref[...] = jnp.dot(x_ref[...], w_ref[...],
                         preferred_element_type=jnp.float32)


def _node_mm(x, w):
    n, k = x.shape
    m = w.shape[1]
    return pl.pallas_call(
        _node_mm_body,
        grid=(n // BN,),
        in_specs=[_rows(BN, k), _full((k, m))],
        out_specs=_rows(BN, m),
        out_shape=jax.ShapeDtypeStruct((n, m), jnp.float32),
    )(x, w)


def _nc_tail_body(h_ref, w_ref, b_ref, o_ref):
    f32 = jnp.float32
    z = jnp.maximum(h_ref[...] + b_ref[0], 0.0)
    for l in range(3):
        z = jnp.maximum(jnp.dot(z, w_ref[l], preferred_element_type=f32)
                        + b_ref[l + 1], 0.0)
    o_ref[...] = (jnp.dot(z, w_ref[3], preferred_element_type=f32)
                  + b_ref[4])


def _nc_tail(h, w_stack, b_stack):
    e = h.shape[0]
    return pl.pallas_call(
        _nc_tail_body,
        grid=(e // BE,),
        in_specs=[_rows(BE, C), _full((4, C, C)), _full((5, 1, C))],
        out_specs=_rows(BE, C),
        out_shape=jax.ShapeDtypeStruct((e, C), jnp.float32),
    )(h, w_stack, b_stack)


def _node_upd_body(x_ref, sa_ref, sb_ref, c_ref, o_ref):
    s = sa_ref[0] + sa_ref[1] + sb_ref[0] + sb_ref[1]
    cnt = jnp.maximum((c_ref[0] + c_ref[1])[:, :1], 1.0)
    o_ref[...] = jnp.maximum(x_ref[...] + s / cnt, 0.0)


def _node_update(x, sa, sb, cnt_part):
    p3 = pl.BlockSpec((2, BN, C), lambda i: (0, i, 0))
    return pl.pallas_call(
        _node_upd_body,
        grid=(N_NODES // BN,),
        in_specs=[_rows(BN, C), p3, p3, p3],
        out_specs=_rows(BN, C),
        out_shape=jax.ShapeDtypeStruct((N_NODES, C), jnp.float32),
    )(x, sa, sb, cnt_part)


def _mid_body(xs_ref, xd_ref, ang_ref,
              ew_ref, eb_ref, wfe_ref, wfa_ref, bf_ref,
              vw1_ref, vw_ref, vb_ref,
              e1_ref, m2_ref, sl_ref):
    f32 = jnp.float32
    xs = xs_ref[...]
    xd = xd_ref[...]
    w1a = ew_ref[0, :C]
    w1b = ew_ref[0, C:]
    hij = jnp.maximum(jnp.dot(xs, w1a, preferred_element_type=f32)
                      + jnp.dot(xd, w1b, preferred_element_type=f32)
                      + eb_ref[0], 0.0)
    hji = jnp.maximum(jnp.dot(xd, w1a, preferred_element_type=f32)
                      + jnp.dot(xs, w1b, preferred_element_type=f32)
                      + eb_ref[0], 0.0)
    for l in range(1, 4):
        w = ew_ref[l, :2 * C]
        hij = jnp.maximum(jnp.dot(hij, w, preferred_element_type=f32)
                          + eb_ref[l], 0.0)
        hji = jnp.maximum(jnp.dot(hji, w, preferred_element_type=f32)
                          + eb_ref[l], 0.0)
    w = ew_ref[4, :2 * C]
    fij = jnp.dot(hij, w, preferred_element_type=f32) + eb_ref[4]
    fji = jnp.dot(hji, w, preferred_element_type=f32) + eb_ref[4]
    d = fij - fji

    @pl.when(pl.program_id(0) == 0)
    def _():
        sl_ref[...] = jnp.zeros((1, 1), jnp.float32)

    sl_ref[...] += jnp.sum(d * d).reshape(1, 1)
    e = 0.5 * (fij + fji)
    e1_ref[...] = jnp.maximum(
        jnp.dot(e, wfe_ref[...], preferred_element_type=f32)
        + ang_ref[...] * wfa_ref[...] + bf_ref[...], 0.0)
    v1a = vw1_ref[:C]
    v1b = vw1_ref[C:]
    g = jnp.maximum(jnp.dot(xs, v1a, preferred_element_type=f32)
                    + jnp.dot(xd, v1b, preferred_element_type=f32)
                    + vb_ref[0], 0.0)
    for l in range(3):
        g = jnp.maximum(jnp.dot(g, vw_ref[l], preferred_element_type=f32)
                        + vb_ref[l + 1], 0.0)
    m2_ref[...] = (jnp.dot(g, vw_ref[3], preferred_element_type=f32)
                   + vb_ref[4])


def _mid(xs, xd, ang, ew, eb, wfe, wfa, bf, vw1, vw, vb):
    e = xs.shape[0]
    return pl.pallas_call(
        _mid_body,
        grid=(e // BE,),
        in_specs=[_rows(BE, C), _rows(BE, C), _rows(BE, 1),
                  _full((5, 2 * C, 2 * C)), _full((5, 1, 2 * C)),
                  _full((2 * C, 2 * C)), _full((1, 2 * C)), _full((1, 2 * C)),
                  _full((2 * C, C)), _full((4, C, C)), _full((5, 1, C))],
        out_specs=[_rows(BE, 2 * C), _rows(BE, C),
                   pl.BlockSpec((1, 1), lambda i: (0, 0))],
        out_shape=[jax.ShapeDtypeStruct((e, 2 * C), jnp.float32),
                   jax.ShapeDtypeStruct((e, C), jnp.float32),
                   jax.ShapeDtypeStruct((1, 1), jnp.float32)],
    )(xs, xd, ang, ew, eb, wfe, wfa, bf, vw1, vw, vb)


def _ec2_body(xs_ref, xd_ref, e1_ref,
              uw1_ref, uw_ref, ub_ref, wfe_ref, wfi_ref, bf_ref,
              e2_ref, sl_ref):
    f32 = jnp.float32
    xs = xs_ref[...]
    xd = xd_ref[...]
    u1a = uw1_ref[:C]
    u1b = uw1_ref[C:]
    hij = jnp.maximum(jnp.dot(xs, u1a, preferred_element_type=f32)
                      + jnp.dot(xd, u1b, preferred_element_type=f32)
                      + ub_ref[0], 0.0)
    hji = jnp.maximum(jnp.dot(xd, u1a, preferred_element_type=f32)
                      + jnp.dot(xs, u1b, preferred_element_type=f32)
                      + ub_ref[0], 0.0)
    for l in range(3):
        w = uw_ref[l]
        hij = jnp.maximum(jnp.dot(hij, w, preferred_element_type=f32)
                          + ub_ref[l + 1], 0.0)
        hji = jnp.maximum(jnp.dot(hji, w, preferred_element_type=f32)
                          + ub_ref[l + 1], 0.0)
    w = uw_ref[3]
    fij = jnp.dot(hij, w, preferred_element_type=f32) + ub_ref[4]
    fji = jnp.dot(hji, w, preferred_element_type=f32) + ub_ref[4]
    d = fij - fji

    @pl.when(pl.program_id(0) == 0)
    def _():
        sl_ref[...] = jnp.zeros((1, 1), jnp.float32)

    sl_ref[...] += jnp.sum(d * d).reshape(1, 1)
    e = 0.5 * (fij + fji)
    e2_ref[...] = jnp.maximum(
        jnp.dot(e, wfe_ref[...], preferred_element_type=f32)
        + jnp.dot(e1_ref[...], wfi_ref[...], preferred_element_type=f32)
        + bf_ref[...], 0.0)


def _ec2(xs, xd, e1, uw1, uw, ub, wfe, wfi, bf):
    e = xs.shape[0]
    return pl.pallas_call(
        _ec2_body,
        grid=(e // BE,),
        in_specs=[_rows(BE, C), _rows(BE, C), _rows(BE, 2 * C),
                  _full((2 * C, C)), _full((4, C, C)), _full((5, 1, C)),
                  _full((C, C)), _full((2 * C, C)), _full((1, C))],
        out_specs=[_rows(BE, C), pl.BlockSpec((1, 1), lambda i: (0, 0))],
        out_shape=[jax.ShapeDtypeStruct((e, C), jnp.float32),
                   jax.ShapeDtypeStruct((1, 1), jnp.float32)],
    )(xs, xd, e1, uw1, uw, ub, wfe, wfi, bf)


def _stack_mlp(layers):
    w1 = layers[0][0]
    wt = jnp.stack([w for (w, _) in layers[1:]])
    bs = jnp.stack([b[None, :] for (_, b) in layers])
    return w1, wt, bs


def _pad_dst2d(dsth, e):
    epw, nch, rem = _geom(e)
    nrows = -(-(nch + (1 if rem else 0)) // 8) * 8
    dstw = dsth.reshape(SC_NW, epw)
    pad = jnp.full((SC_NW, nrows * CH - epw), NPAD - 1, jnp.int32)
    return jnp.concatenate([dstw, pad], axis=1).reshape(-1, CH)


def kernel(node_features, edge_index, angles, gt_edges,
           nc1, ec1_mlp, ec1_fuse, nc2, ec2_mlp, ec2_fuse):
    src, dst = edge_index[0], edge_index[1]
    x0 = node_features
    halves = ((E1, src[:E1], dst[:E1], angles[:E1]),
              (E2, src[E1:], dst[E1:], angles[E1:]))
    dst2d = tuple(_pad_dst2d(h[2], h[0]) for h in halves)
    gath = {e: _make_sc_gather(C, e) for e in (E1, E2)}
    gath2 = {e: _make_sc_gather2(C, e) for e in (E1, E2)}
    seg = {e: _make_sc_segsum(e) for e in (E1, E2)}
    cntp = _make_sc_count()(dst)

    n1w1, n1wt, n1b = _stack_mlp(nc1)
    ab1 = _node_mm(x0, jnp.concatenate([n1w1[:C], n1w1[C:]], axis=1))
    m1p = []
    for (e, s_, d_, _), d2 in zip(halves, dst2d):
        h1 = gath2[e](ab1[:, :C], ab1[:, C:], s_, d_)
        m1p.append(seg[e](_nc_tail(h1, n1wt, n1b), d2))
    x1 = _node_update(x0, m1p[0], m1p[1], cntp)

    ew = jnp.stack([w for (w, _) in ec1_mlp])
    eb = jnp.stack([b[None, :] for (_, b) in ec1_mlp])
    wf1, bf1 = ec1_fuse
    n2w1, n2wt, n2b = _stack_mlp(nc2)
    e1h, s2p, sl1s = [], [], []
    for (e, s_, d_, ang), d2 in zip(halves, dst2d):
        xs1 = gath[e](x1, s_)
        xd1 = gath[e](x1, d_)
        e1_, m2_, sl_ = _mid(xs1, xd1, ang, ew, eb,
                             wf1[:2 * C], wf1[2 * C:2 * C + 1], bf1[None, :],
                             n2w1, n2wt, n2b)
        e1h.append(e1_)
        s2p.append(seg[e](m2_, d2))
        sl1s.append(sl_)
    x2 = _node_update(x1, s2p[0], s2p[1], cntp)

    e2w1, e2wt, e2b = _stack_mlp(ec2_mlp)
    wf2, bf2 = ec2_fuse
    e2h, sl2s = [], []
    for (e, s_, d_, _), e1_ in zip(halves, e1h):
        xs2 = gath[e](x2, s_)
        xd2 = gath[e](x2, d_)
        e2_, sl_ = _ec2(xs2, xd2, e1_, e2w1, e2wt, e2b,
                        wf2[:C], wf2[C:], bf2[None, :])
        e2h.append(e2_)
        sl2s.append(sl_)

    e2 = jnp.concatenate(e2h, axis=0)
    sl1 = (sl1s[0][0, 0] + sl1s[1][0, 0]) / (N_EDGES * 2 * C)
    sl2 = (sl2s[0][0, 0] + sl2s[1][0, 0]) / (N_EDGES * C)
    return (e2, (sl1 + sl2) / 2.0)


# speedup vs baseline: 1.1747x; 1.0126x over previous
import functools

import jax
import jax.numpy as jnp
from jax import lax
from jax.experimental import pallas as pl
from jax.experimental.pallas import tpu as pltpu
from jax.experimental.pallas import tpu_sc as plsc

N_NODES = 10000
N_EDGES = 160000
C = 128
BE = 2000
BN = 2000

SC_NC = 2
SC_NS = 16
SC_NW = SC_NC * SC_NS
CH = 128
NPAD = 10240
NPA = NPAD // SC_NS


def _geom(e):
    epw = e // SC_NW
    return epw, epw // CH, epw % CH


def _sc_mesh():
    return plsc.VectorSubcoreMesh(core_axis_name="c", subcore_axis_name="s",
                                  num_cores=SC_NC, num_subcores=SC_NS)


def _wid():
    return lax.axis_index("s") * SC_NC + lax.axis_index("c")


def _pipe(nch, gath, fin):
    gath(0, 0)

    def body(t, _):
        j = t * 2
        gath(j + 1, 1)
        fin(j, 0)
        gath(j + 2, 0)
        fin(j + 1, 1)
        return _

    lax.fori_loop(0, (nch - 1) // 2, body, 0)
    if nch % 2:
        fin(nch - 1, 0)
    else:
        gath(nch - 1, 1)
        fin(nch - 2, 0)
        fin(nch - 1, 1)


def _make_sc_gather(d, e):
    epw, nch, rem = _geom(e)

    @functools.partial(
        pl.kernel,
        out_type=jax.ShapeDtypeStruct((e, d), jnp.float32),
        mesh=_sc_mesh(),
        scratch_types=[pltpu.VMEM((epw,), jnp.int32),
                       pltpu.VMEM((CH, d), jnp.float32),
                       pltpu.VMEM((CH, d), jnp.float32),
                       pltpu.SemaphoreType.DMA,
                       pltpu.SemaphoreType.DMA],
    )
    def k(table_hbm, idx_hbm, out_hbm, idx_v, rows0, rows1, sem0, sem1):
        base = _wid() * epw
        pltpu.sync_copy(idx_hbm.at[pl.ds(base, epw)], idx_v)
        bufs = (rows0, rows1)
        sems = (sem0, sem1)

        def gath(j, p):
            pltpu.async_copy(
                table_hbm.at[idx_v.at[pl.ds(j * CH, CH)]], bufs[p], sems[p])

        def fin(j, p):
            pltpu.make_async_copy(
                table_hbm.at[pl.ds(0, CH)], bufs[p], sems[p]).wait()
            pltpu.sync_copy(bufs[p], out_hbm.at[pl.ds(base + j * CH, CH)])

        _pipe(nch, gath, fin)
        if rem:
            off = nch * CH
            pltpu.async_copy(table_hbm.at[idx_v.at[pl.ds(off, rem)]],
                             rows0.at[pl.ds(0, rem)], sem0).wait()
            pltpu.sync_copy(rows0.at[pl.ds(0, rem)],
                            out_hbm.at[pl.ds(base + off, rem)])

    return k


def _make_sc_gather2(d, e):
    epw, nch, rem = _geom(e)

    @functools.partial(
        pl.kernel,
        out_type=jax.ShapeDtypeStruct((e, d), jnp.float32),
        mesh=_sc_mesh(),
        scratch_types=[pltpu.VMEM((epw,), jnp.int32),
                       pltpu.VMEM((epw,), jnp.int32),
                       pltpu.VMEM((CH, d), jnp.float32),
                       pltpu.VMEM((CH, d), jnp.float32),
                       pltpu.SemaphoreType.DMA,
                       pltpu.SemaphoreType.DMA],
    )
    def k(ta_hbm, tb_hbm, ia_hbm, ib_hbm, out_hbm,
          ia_v, ib_v, rows0, rows1, sem0, sem1):
        base = _wid() * epw
        pltpu.sync_copy(ia_hbm.at[pl.ds(base, epw)], ia_v)
        pltpu.sync_copy(ib_hbm.at[pl.ds(base, epw)], ib_v)
        bufs = (rows0, rows1)
        sems = (sem0, sem1)

        def gath(j, p):
            pltpu.async_copy(
                ta_hbm.at[ia_v.at[pl.ds(j * CH, CH)]], bufs[p], sems[p])

        def fin(j, p):
            pltpu.make_async_copy(
                ta_hbm.at[pl.ds(0, CH)], bufs[p], sems[p]).wait()
            pltpu.sync_copy(tb_hbm.at[ib_v.at[pl.ds(j * CH, CH)]], bufs[p],
                            add=True)
            pltpu.sync_copy(bufs[p], out_hbm.at[pl.ds(base + j * CH, CH)])

        _pipe(nch, gath, fin)
        if rem:
            off = nch * CH
            pltpu.async_copy(ta_hbm.at[ia_v.at[pl.ds(off, rem)]],
                             rows0.at[pl.ds(0, rem)], sem0).wait()
            pltpu.sync_copy(tb_hbm.at[ib_v.at[pl.ds(off, rem)]],
                            rows0.at[pl.ds(0, rem)], add=True)
            pltpu.sync_copy(rows0.at[pl.ds(0, rem)],
                            out_hbm.at[pl.ds(base + off, rem)])

    return k


def _make_sc_segsum(e):
    epw, nch, rem = _geom(e)
    nrows = -(-(nch + (1 if rem else 0)) // 8) * 8

    @functools.partial(
        pl.kernel,
        out_type=jax.ShapeDtypeStruct((SC_NC, NPAD, C), jnp.float32),
        mesh=_sc_mesh(),
        scratch_types=[pltpu.VMEM((nrows, CH), jnp.int32),
                       pltpu.VMEM((CH, C), jnp.float32),
                       pltpu.VMEM((CH, C), jnp.float32),
                       pltpu.VMEM_SHARED((NPAD, C), jnp.float32),
                       pltpu.SemaphoreType.DMA,
                       pltpu.SemaphoreType.DMA],
    )
    def k(m_hbm, dst2d_hbm, out_hbm, idx_v, rows0, rows1, acc_sh,
          sem0, sem1):
        cid = lax.axis_index("c")
        sid = lax.axis_index("s")
        wid = _wid()
        base = wid * epw

        def zrow(i, _):
            for t in range(C // 16):
                rows0[i, pl.ds(t * 16, 16)] = jnp.zeros((16,), jnp.float32)
            return _

        lax.fori_loop(0, CH, zrow, 0)
        row0 = sid * NPA
        for t in range(NPA // CH):
            pltpu.sync_copy(rows0, acc_sh.at[pl.ds(row0 + t * CH, CH)])
        pltpu.sync_copy(dst2d_hbm.at[pl.ds(wid * nrows, nrows)], idx_v)
        plsc.subcore_barrier()

        bufs = (rows0, rows1)
        sems = (sem0, sem1)

        def load(j, p):
            pltpu.async_copy(m_hbm.at[pl.ds(base + j * CH, CH)], bufs[p],
                             sems[p])

        def fin(j, p):
            pltpu.make_async_copy(m_hbm.at[pl.ds(0, CH)], bufs[p],
                                  sems[p]).wait()
            pltpu.sync_copy(bufs[p], acc_sh.at[idx_v.at[j]], add=True)

        _pipe(nch, load, fin)
        if rem:
            pltpu.sync_copy(m_hbm.at[pl.ds(base + nch * CH, rem)],
                            rows1.at[pl.ds(0, rem)])
            pltpu.sync_copy(rows1, acc_sh.at[idx_v.at[nch]], add=True)
        plsc.subcore_barrier()

        pltpu.sync_copy(acc_sh.at[pl.ds(row0, NPA)],
                        out_hbm.at[cid].at[pl.ds(row0, NPA)])

    return k


def _make_sc_count():
    epw, nch, rem = _geom(N_EDGES)

    @functools.partial(
        pl.kernel,
        out_type=jax.ShapeDtypeStruct((SC_NC, NPAD, C), jnp.float32),
        mesh=_sc_mesh(),
        scratch_types=[pltpu.VMEM((CH,), jnp.int32),
                       pltpu.VMEM((rem,), jnp.int32),
                       pltpu.VMEM((CH, C), jnp.float32),
                       pltpu.VMEM((CH, C), jnp.float32),
                       pltpu.VMEM_SHARED((NPAD, C), jnp.float32),
                       pltpu.SemaphoreType.DMA],
    )
    def k(dst_hbm, out_hbm, idx_v, idx8_v, ones_v, zc_v, acc_sh, sem1):
        cid = lax.axis_index("c")
        sid = lax.axis_index("s")
        base = _wid() * epw

        def orow(i, _):
            for t in range(C // 16):
                ones_v[i, pl.ds(t * 16, 16)] = jnp.ones((16,), jnp.float32)
                zc_v[i, pl.ds(t * 16, 16)] = jnp.zeros((16,), jnp.float32)
            return _

        lax.fori_loop(0, CH, orow, 0)
        row0 = sid * NPA
        for t in range(NPA // CH):
            pltpu.sync_copy(zc_v, acc_sh.at[pl.ds(row0 + t * CH, CH)])
        plsc.subcore_barrier()

        def body(j, _):
            off = base + j * CH
            pltpu.async_copy(dst_hbm.at[pl.ds(off, CH)], idx_v, sem1).wait()
            pltpu.sync_copy(ones_v, acc_sh.at[idx_v], add=True)
            return _

        lax.fori_loop(0, nch, body, 0)
        off = base + nch * CH
        pltpu.sync_copy(dst_hbm.at[pl.ds(off, rem)], idx8_v)
        pltpu.sync_copy(ones_v.at[pl.ds(0, rem)], acc_sh.at[idx8_v],
                        add=True)
        plsc.subcore_barrier()

        pltpu.sync_copy(acc_sh.at[pl.ds(row0, NPA)],
                        out_hbm.at[cid].at[pl.ds(row0, NPA)])

    return k


def _full(shape):
    return pl.BlockSpec(shape, lambda i: (0,) * len(shape))


def _rows(bs, width):
    return pl.BlockSpec((bs, width), lambda i: (i, 0))


def _node_mm_body(x_ref, w_ref, o_ref):
    o_ref[...] = jnp.dot(x_ref[...], w_ref[...],
                         preferred_element_type=jnp.float32)


def _node_mm(x, w):
    n, k = x.shape
    m = w.shape[1]
    return pl.pallas_call(
        _node_mm_body,
        grid=(n // BN,),
        in_specs=[_rows(BN, k), _full((k, m))],
        out_specs=_rows(BN, m),
        out_shape=jax.ShapeDtypeStruct((n, m), jnp.float32),
    )(x, w)


def _nc_tail_body(h_ref, w_ref, b_ref, o_ref):
    f32 = jnp.float32
    z = jnp.maximum(h_ref[...] + b_ref[0], 0.0)
    for l in range(3):
        z = jnp.maximum(jnp.dot(z, w_ref[l], preferred_element_type=f32)
                        + b_ref[l + 1], 0.0)
    o_ref[...] = (jnp.dot(z, w_ref[3], preferred_element_type=f32)
                  + b_ref[4])


def _nc_tail(h, w_stack, b_stack):
    e = h.shape[0]
    return pl.pallas_call(
        _nc_tail_body,
        grid=(e // BE,),
        in_specs=[_rows(BE, C), _full((4, C, C)), _full((5, 1, C))],
        out_specs=_rows(BE, C),
        out_shape=jax.ShapeDtypeStruct((e, C), jnp.float32),
    )(h, w_stack, b_stack)


def _node_upd_body(x_ref, s_ref, c_ref, o_ref):
    s = s_ref[0] + s_ref[1]
    cnt = jnp.maximum((c_ref[0] + c_ref[1])[:, :1], 1.0)
    o_ref[...] = jnp.maximum(x_ref[...] + s / cnt, 0.0)


def _node_update(x, s_part, cnt_part):
    p3 = pl.BlockSpec((2, BN, C), lambda i: (0, i, 0))
    return pl.pallas_call(
        _node_upd_body,
        grid=(N_NODES // BN,),
        in_specs=[_rows(BN, C), p3, p3],
        out_specs=_rows(BN, C),
        out_shape=jax.ShapeDtypeStruct((N_NODES, C), jnp.float32),
    )(x, s_part, cnt_part)


def _mid_body(xs_ref, xd_ref, ang_ref,
              ew_ref, eb_ref, wfe_ref, wfa_ref, bf_ref,
              vw1_ref, vw_ref, vb_ref,
              e1_ref, m2_ref, sl_ref):
    f32 = jnp.float32
    xs = xs_ref[...]
    xd = xd_ref[...]
    w1a = ew_ref[0, :C]
    w1b = ew_ref[0, C:]
    hij = jnp.maximum(jnp.dot(xs, w1a, preferred_element_type=f32)
                      + jnp.dot(xd, w1b, preferred_element_type=f32)
                      + eb_ref[0], 0.0)
    hji = jnp.maximum(jnp.dot(xd, w1a, preferred_element_type=f32)
                      + jnp.dot(xs, w1b, preferred_element_type=f32)
                      + eb_ref[0], 0.0)
    for l in range(1, 4):
        w = ew_ref[l, :2 * C]
        hij = jnp.maximum(jnp.dot(hij, w, preferred_element_type=f32)
                          + eb_ref[l], 0.0)
        hji = jnp.maximum(jnp.dot(hji, w, preferred_element_type=f32)
                          + eb_ref[l], 0.0)
    w = ew_ref[4, :2 * C]
    fij = jnp.dot(hij, w, preferred_element_type=f32) + eb_ref[4]
    fji = jnp.dot(hji, w, preferred_element_type=f32) + eb_ref[4]
    d = fij - fji

    @pl.when(pl.program_id(0) == 0)
    def _():
        sl_ref[...] = jnp.zeros((1, 1), jnp.float32)

    sl_ref[...] += jnp.sum(d * d).reshape(1, 1)
    e = 0.5 * (fij + fji)
    e1_ref[...] = jnp.maximum(
        jnp.dot(e, wfe_ref[...], preferred_element_type=f32)
        + ang_ref[...] * wfa_ref[...] + bf_ref[...], 0.0)
    v1a = vw1_ref[:C]
    v1b = vw1_ref[C:]
    g = jnp.maximum(jnp.dot(xs, v1a, preferred_element_type=f32)
                    + jnp.dot(xd, v1b, preferred_element_type=f32)
                    + vb_ref[0], 0.0)
    for l in range(3):
        g = jnp.maximum(jnp.dot(g, vw_ref[l], preferred_element_type=f32)
                        + vb_ref[l + 1], 0.0)
    m2_ref[...] = (jnp.dot(g, vw_ref[3], preferred_element_type=f32)
                   + vb_ref[4])


def _mid(xs, xd, ang, ew, eb, wfe, wfa, bf, vw1, vw, vb):
    e = xs.shape[0]
    return pl.pallas_call(
        _mid_body,
        grid=(e // BE,),
        in_specs=[_rows(BE, C), _rows(BE, C), _rows(BE, 1),
                  _full((5, 2 * C, 2 * C)), _full((5, 1, 2 * C)),
                  _full((2 * C, 2 * C)), _full((1, 2 * C)), _full((1, 2 * C)),
                  _full((2 * C, C)), _full((4, C, C)), _full((5, 1, C))],
        out_specs=[_rows(BE, 2 * C), _rows(BE, C),
                   pl.BlockSpec((1, 1), lambda i: (0, 0))],
        out_shape=[jax.ShapeDtypeStruct((e, 2 * C), jnp.float32),
                   jax.ShapeDtypeStruct((e, C), jnp.float32),
                   jax.ShapeDtypeStruct((1, 1), jnp.float32)],
    )(xs, xd, ang, ew, eb, wfe, wfa, bf, vw1, vw, vb)


def _ec2_body(xs_ref, xd_ref, e1_ref,
              uw1_ref, uw_ref, ub_ref, wfe_ref, wfi_ref, bf_ref,
              e2_ref, sl_ref):
    f32 = jnp.float32
    xs = xs_ref[...]
    xd = xd_ref[...]
    u1a = uw1_ref[:C]
    u1b = uw1_ref[C:]
    hij = jnp.maximum(jnp.dot(xs, u1a, preferred_element_type=f32)
                      + jnp.dot(xd, u1b, preferred_element_type=f32)
                      + ub_ref[0], 0.0)
    hji = jnp.maximum(jnp.dot(xd, u1a, preferred_element_type=f32)
                      + jnp.dot(xs, u1b, preferred_element_type=f32)
                      + ub_ref[0], 0.0)
    for l in range(3):
        w = uw_ref[l]
        hij = jnp.maximum(jnp.dot(hij, w, preferred_element_type=f32)
                          + ub_ref[l + 1], 0.0)
        hji = jnp.maximum(jnp.dot(hji, w, preferred_element_type=f32)
                          + ub_ref[l + 1], 0.0)
    w = uw_ref[3]
    fij = jnp.dot(hij, w, preferred_element_type=f32) + ub_ref[4]
    fji = jnp.dot(hji, w, preferred_element_type=f32) + ub_ref[4]
    d = fij - fji

    @pl.when(pl.program_id(0) == 0)
    def _():
        sl_ref[...] = jnp.zeros((1, 1), jnp.float32)

    sl_ref[...] += jnp.sum(d * d).reshape(1, 1)
    e = 0.5 * (fij + fji)
    e2_ref[...] = jnp.maximum(
        jnp.dot(e, wfe_ref[...], preferred_element_type=f32)
        + jnp.dot(e1_ref[...], wfi_ref[...], preferred_element_type=f32)
        + bf_ref[...], 0.0)


def _ec2(xs, xd, e1, uw1, uw, ub, wfe, wfi, bf):
    e = xs.shape[0]
    return pl.pallas_call(
        _ec2_body,
        grid=(e // BE,),
        in_specs=[_rows(BE, C), _rows(BE, C), _rows(BE, 2 * C),
                  _full((2 * C, C)), _full((4, C, C)), _full((5, 1, C)),
                  _full((C, C)), _full((2 * C, C)), _full((1, C))],
        out_specs=[_rows(BE, C), pl.BlockSpec((1, 1), lambda i: (0, 0))],
        out_shape=[jax.ShapeDtypeStruct((e, C), jnp.float32),
                   jax.ShapeDtypeStruct((1, 1), jnp.float32)],
    )(xs, xd, e1, uw1, uw, ub, wfe, wfi, bf)


def _stack_mlp(layers):
    w1 = layers[0][0]
    wt = jnp.stack([w for (w, _) in layers[1:]])
    bs = jnp.stack([b[None, :] for (_, b) in layers])
    return w1, wt, bs


def _pad_dst2d(dsth, e):
    epw, nch, rem = _geom(e)
    nrows = -(-(nch + (1 if rem else 0)) // 8) * 8
    dstw = dsth.reshape(SC_NW, epw)
    pad = jnp.full((SC_NW, nrows * CH - epw), NPAD - 1, jnp.int32)
    return jnp.concatenate([dstw, pad], axis=1).reshape(-1, CH)


def kernel(node_features, edge_index, angles, gt_edges,
           nc1, ec1_mlp, ec1_fuse, nc2, ec2_mlp, ec2_fuse):
    src, dst = edge_index[0], edge_index[1]
    x0 = node_features
    dst2d = _pad_dst2d(dst, N_EDGES)
    gath = _make_sc_gather(C, N_EDGES)
    gath2 = _make_sc_gather2(C, N_EDGES)
    seg = _make_sc_segsum(N_EDGES)
    cntp = _make_sc_count()(dst)

    n1w1, n1wt, n1b = _stack_mlp(nc1)
    ab1 = _node_mm(x0, jnp.concatenate([n1w1[:C], n1w1[C:]], axis=1))
    h1 = gath2(ab1[:, :C], ab1[:, C:], src, dst)
    s1p = seg(_nc_tail(h1, n1wt, n1b), dst2d)
    x1 = _node_update(x0, s1p, cntp)

    ew = jnp.stack([w for (w, _) in ec1_mlp])
    eb = jnp.stack([b[None, :] for (_, b) in ec1_mlp])
    wf1, bf1 = ec1_fuse
    n2w1, n2wt, n2b = _stack_mlp(nc2)
    xs1 = gath(x1, src)
    xd1 = gath(x1, dst)
    e1, m2, sl1s = _mid(xs1, xd1, angles, ew, eb,
                        wf1[:2 * C], wf1[2 * C:2 * C + 1], bf1[None, :],
                        n2w1, n2wt, n2b)
    s2p = seg(m2, dst2d)
    x2 = _node_update(x1, s2p, cntp)

    e2w1, e2wt, e2b = _stack_mlp(ec2_mlp)
    wf2, bf2 = ec2_fuse
    xs2 = gath(x2, src)
    xd2 = gath(x2, dst)
    e2, sl2s = _ec2(xs2, xd2, e1, e2w1, e2wt, e2b,
                    wf2[:C], wf2[C:], bf2[None, :])

    sl1 = sl1s[0, 0] / (N_EDGES * 2 * C)
    sl2 = sl2s[0, 0] / (N_EDGES * C)
    return (e2, (sl1 + sl2) / 2.0)


# speedup vs baseline: 1.2221x; 1.0404x over previous
import functools

import jax
import jax.numpy as jnp
from jax import lax
from jax.experimental import pallas as pl
from jax.experimental.pallas import tpu as pltpu
from jax.experimental.pallas import tpu_sc as plsc

N_NODES = 10000
N_EDGES = 160000
C = 128
BE = 4000
BN = 2000

SC_NC = 2
SC_NS = 16
SC_NW = SC_NC * SC_NS
CH = 128
NPAD = 10240
NPA = NPAD // SC_NS


def _geom(e):
    epw = e // SC_NW
    return epw, epw // CH, epw % CH


def _sc_mesh():
    return plsc.VectorSubcoreMesh(core_axis_name="c", subcore_axis_name="s",
                                  num_cores=SC_NC, num_subcores=SC_NS)


def _wid():
    return lax.axis_index("s") * SC_NC + lax.axis_index("c")


def _pipe(nch, gath, fin):
    gath(0, 0)

    def body(t, _):
        j = t * 2
        gath(j + 1, 1)
        fin(j, 0)
        gath(j + 2, 0)
        fin(j + 1, 1)
        return _

    lax.fori_loop(0, (nch - 1) // 2, body, 0)
    if nch % 2:
        fin(nch - 1, 0)
    else:
        gath(nch - 1, 1)
        fin(nch - 2, 0)
        fin(nch - 1, 1)


def _make_sc_gather(d, e):
    epw, nch, rem = _geom(e)

    @functools.partial(
        pl.kernel,
        out_type=jax.ShapeDtypeStruct((e, d), jnp.float32),
        mesh=_sc_mesh(),
        scratch_types=[pltpu.VMEM((epw,), jnp.int32),
                       pltpu.VMEM((CH, d), jnp.float32),
                       pltpu.VMEM((CH, d), jnp.float32),
                       pltpu.SemaphoreType.DMA,
                       pltpu.SemaphoreType.DMA],
    )
    def k(table_hbm, idx_hbm, out_hbm, idx_v, rows0, rows1, sem0, sem1):
        base = _wid() * epw
        pltpu.sync_copy(idx_hbm.at[pl.ds(base, epw)], idx_v)
        bufs = (rows0, rows1)
        sems = (sem0, sem1)

        def gath(j, p):
            pltpu.async_copy(
                table_hbm.at[idx_v.at[pl.ds(j * CH, CH)]], bufs[p], sems[p])

        def fin(j, p):
            pltpu.make_async_copy(
                table_hbm.at[pl.ds(0, CH)], bufs[p], sems[p]).wait()
            pltpu.sync_copy(bufs[p], out_hbm.at[pl.ds(base + j * CH, CH)])

        _pipe(nch, gath, fin)
        if rem:
            off = nch * CH
            pltpu.async_copy(table_hbm.at[idx_v.at[pl.ds(off, rem)]],
                             rows0.at[pl.ds(0, rem)], sem0).wait()
            pltpu.sync_copy(rows0.at[pl.ds(0, rem)],
                            out_hbm.at[pl.ds(base + off, rem)])

    return k


def _make_sc_gather2(d, e):
    epw, nch, rem = _geom(e)

    @functools.partial(
        pl.kernel,
        out_type=jax.ShapeDtypeStruct((e, d), jnp.float32),
        mesh=_sc_mesh(),
        scratch_types=[pltpu.VMEM((epw,), jnp.int32),
                       pltpu.VMEM((epw,), jnp.int32),
                       pltpu.VMEM((CH, d), jnp.float32),
                       pltpu.VMEM((CH, d), jnp.float32),
                       pltpu.SemaphoreType.DMA,
                       pltpu.SemaphoreType.DMA],
    )
    def k(ta_hbm, tb_hbm, ia_hbm, ib_hbm, out_hbm,
          ia_v, ib_v, rows0, rows1, sem0, sem1):
        base = _wid() * epw
        pltpu.sync_copy(ia_hbm.at[pl.ds(base, epw)], ia_v)
        pltpu.sync_copy(ib_hbm.at[pl.ds(base, epw)], ib_v)
        bufs = (rows0, rows1)
        sems = (sem0, sem1)

        def gath(j, p):
            pltpu.async_copy(
                ta_hbm.at[ia_v.at[pl.ds(j * CH, CH)]], bufs[p], sems[p])

        def fin(j, p):
            pltpu.make_async_copy(
                ta_hbm.at[pl.ds(0, CH)], bufs[p], sems[p]).wait()
            pltpu.sync_copy(tb_hbm.at[ib_v.at[pl.ds(j * CH, CH)]], bufs[p],
                            add=True)
            pltpu.sync_copy(bufs[p], out_hbm.at[pl.ds(base + j * CH, CH)])

        _pipe(nch, gath, fin)
        if rem:
            off = nch * CH
            pltpu.async_copy(ta_hbm.at[ia_v.at[pl.ds(off, rem)]],
                             rows0.at[pl.ds(0, rem)], sem0).wait()
            pltpu.sync_copy(tb_hbm.at[ib_v.at[pl.ds(off, rem)]],
                            rows0.at[pl.ds(0, rem)], add=True)
            pltpu.sync_copy(rows0.at[pl.ds(0, rem)],
                            out_hbm.at[pl.ds(base + off, rem)])

    return k


def _make_sc_segsum(e):
    epw, nch, rem = _geom(e)
    nrows = -(-(nch + (1 if rem else 0)) // 8) * 8

    @functools.partial(
        pl.kernel,
        out_type=jax.ShapeDtypeStruct((SC_NC, NPAD, C), jnp.float32),
        mesh=_sc_mesh(),
        scratch_types=[pltpu.VMEM((nrows, CH), jnp.int32),
                       pltpu.VMEM((CH, C), jnp.float32),
                       pltpu.VMEM((CH, C), jnp.float32),
                       pltpu.VMEM_SHARED((NPAD, C), jnp.float32),
                       pltpu.SemaphoreType.DMA,
                       pltpu.SemaphoreType.DMA],
    )
    def k(m_hbm, dst2d_hbm, out_hbm, idx_v, rows0, rows1, acc_sh,
          sem0, sem1):
        cid = lax.axis_index("c")
        sid = lax.axis_index("s")
        wid = _wid()
        base = wid * epw

        def zrow(i, _):
            for t in range(C // 16):
                rows0[i, pl.ds(t * 16, 16)] = jnp.zeros((16,), jnp.float32)
            return _

        lax.fori_loop(0, CH, zrow, 0)
        row0 = sid * NPA
        for t in range(NPA // CH):
            pltpu.sync_copy(rows0, acc_sh.at[pl.ds(row0 + t * CH, CH)])
        pltpu.sync_copy(dst2d_hbm.at[pl.ds(wid * nrows, nrows)], idx_v)
        plsc.subcore_barrier()

        bufs = (rows0, rows1)
        sems = (sem0, sem1)

        def load(j, p):
            pltpu.async_copy(m_hbm.at[pl.ds(base + j * CH, CH)], bufs[p],
                             sems[p])

        def fin(j, p):
            pltpu.make_async_copy(m_hbm.at[pl.ds(0, CH)], bufs[p],
                                  sems[p]).wait()
            pltpu.sync_copy(bufs[p], acc_sh.at[idx_v.at[j]], add=True)

        _pipe(nch, load, fin)
        if rem:
            pltpu.sync_copy(m_hbm.at[pl.ds(base + nch * CH, rem)],
                            rows1.at[pl.ds(0, rem)])
            pltpu.sync_copy(rows1, acc_sh.at[idx_v.at[nch]], add=True)
        plsc.subcore_barrier()

        pltpu.sync_copy(acc_sh.at[pl.ds(row0, NPA)],
                        out_hbm.at[cid].at[pl.ds(row0, NPA)])

    return k


def _make_sc_count():
    epw, nch, rem = _geom(N_EDGES)

    @functools.partial(
        pl.kernel,
        out_type=jax.ShapeDtypeStruct((SC_NC, NPAD, C), jnp.float32),
        mesh=_sc_mesh(),
        scratch_types=[pltpu.VMEM((CH,), jnp.int32),
                       pltpu.VMEM((rem,), jnp.int32),
                       pltpu.VMEM((CH, C), jnp.float32),
                       pltpu.VMEM((CH, C), jnp.float32),
                       pltpu.VMEM_SHARED((NPAD, C), jnp.float32),
                       pltpu.SemaphoreType.DMA],
    )
    def k(dst_hbm, out_hbm, idx_v, idx8_v, ones_v, zc_v, acc_sh, sem1):
        cid = lax.axis_index("c")
        sid = lax.axis_index("s")
        base = _wid() * epw

        def orow(i, _):
            for t in range(C // 16):
                ones_v[i, pl.ds(t * 16, 16)] = jnp.ones((16,), jnp.float32)
                zc_v[i, pl.ds(t * 16, 16)] = jnp.zeros((16,), jnp.float32)
            return _

        lax.fori_loop(0, CH, orow, 0)
        row0 = sid * NPA
        for t in range(NPA // CH):
            pltpu.sync_copy(zc_v, acc_sh.at[pl.ds(row0 + t * CH, CH)])
        plsc.subcore_barrier()

        def body(j, _):
            off = base + j * CH
            pltpu.async_copy(dst_hbm.at[pl.ds(off, CH)], idx_v, sem1).wait()
            pltpu.sync_copy(ones_v, acc_sh.at[idx_v], add=True)
            return _

        lax.fori_loop(0, nch, body, 0)
        off = base + nch * CH
        pltpu.sync_copy(dst_hbm.at[pl.ds(off, rem)], idx8_v)
        pltpu.sync_copy(ones_v.at[pl.ds(0, rem)], acc_sh.at[idx8_v],
                        add=True)
        plsc.subcore_barrier()

        pltpu.sync_copy(acc_sh.at[pl.ds(row0, NPA)],
                        out_hbm.at[cid].at[pl.ds(row0, NPA)])

    return k


def _full(shape):
    return pl.BlockSpec(shape, lambda i: (0,) * len(shape))


def _rows(bs, width):
    return pl.BlockSpec((bs, width), lambda i: (i, 0))


def _node_mm_body(x_ref, w_ref, o_ref):
    o_ref[...] = jnp.dot(x_ref[...], w_ref[...],
                         preferred_element_type=jnp.float32)


def _node_mm(x, w):
    n, k = x.shape
    m = w.shape[1]
    return pl.pallas_call(
        _node_mm_body,
        grid=(n // BN,),
        in_specs=[_rows(BN, k), _full((k, m))],
        out_specs=_rows(BN, m),
        out_shape=jax.ShapeDtypeStruct((n, m), jnp.float32),
    )(x, w)


def _nc_tail_body(h_ref, w_ref, b_ref, o_ref):
    f32 = jnp.float32
    z = jnp.maximum(h_ref[...] + b_ref[0], 0.0)
    for l in range(3):
        z = jnp.maximum(jnp.dot(z, w_ref[l], preferred_element_type=f32)
                        + b_ref[l + 1], 0.0)
    o_ref[...] = (jnp.dot(z, w_ref[3], preferred_element_type=f32)
                  + b_ref[4])


def _nc_tail(h, w_stack, b_stack):
    e = h.shape[0]
    return pl.pallas_call(
        _nc_tail_body,
        grid=(e // BE,),
        in_specs=[_rows(BE, C), _full((4, C, C)), _full((5, 1, C))],
        out_specs=_rows(BE, C),
        out_shape=jax.ShapeDtypeStruct((e, C), jnp.float32),
    )(h, w_stack, b_stack)


def _node_upd_body(x_ref, s_ref, c_ref, o_ref):
    s = s_ref[0] + s_ref[1]
    cnt = jnp.maximum((c_ref[0] + c_ref[1])[:, :1], 1.0)
    o_ref[...] = jnp.maximum(x_ref[...] + s / cnt, 0.0)


def _node_update(x, s_part, cnt_part):
    p3 = pl.BlockSpec((2, BN, C), lambda i: (0, i, 0))
    return pl.pallas_call(
        _node_upd_body,
        grid=(N_NODES // BN,),
        in_specs=[_rows(BN, C), p3, p3],
        out_specs=_rows(BN, C),
        out_shape=jax.ShapeDtypeStruct((N_NODES, C), jnp.float32),
    )(x, s_part, cnt_part)


def _mid_body(xs_ref, xd_ref, ang_ref,
              ew_ref, eb_ref, wfe_ref, wfa_ref, bf_ref,
              vw1_ref, vw_ref, vb_ref,
              e1_ref, m2_ref, sl_ref):
    f32 = jnp.float32
    xs = xs_ref[...]
    xd = xd_ref[...]
    w1a = ew_ref[0, :C]
    w1b = ew_ref[0, C:]
    hij = jnp.maximum(jnp.dot(xs, w1a, preferred_element_type=f32)
                      + jnp.dot(xd, w1b, preferred_element_type=f32)
                      + eb_ref[0], 0.0)
    hji = jnp.maximum(jnp.dot(xd, w1a, preferred_element_type=f32)
                      + jnp.dot(xs, w1b, preferred_element_type=f32)
                      + eb_ref[0], 0.0)
    for l in range(1, 4):
        w = ew_ref[l, :2 * C]
        hij = jnp.maximum(jnp.dot(hij, w, preferred_element_type=f32)
                          + eb_ref[l], 0.0)
        hji = jnp.maximum(jnp.dot(hji, w, preferred_element_type=f32)
                          + eb_ref[l], 0.0)
    w = ew_ref[4, :2 * C]
    fij = jnp.dot(hij, w, preferred_element_type=f32) + eb_ref[4]
    fji = jnp.dot(hji, w, preferred_element_type=f32) + eb_ref[4]
    d = fij - fji

    @pl.when(pl.program_id(0) == 0)
    def _():
        sl_ref[...] = jnp.zeros((1, 1), jnp.float32)

    sl_ref[...] += jnp.sum(d * d).reshape(1, 1)
    e = 0.5 * (fij + fji)
    e1_ref[...] = jnp.maximum(
        jnp.dot(e, wfe_ref[...], preferred_element_type=f32)
        + ang_ref[...] * wfa_ref[...] + bf_ref[...], 0.0)
    v1a = vw1_ref[:C]
    v1b = vw1_ref[C:]
    g = jnp.maximum(jnp.dot(xs, v1a, preferred_element_type=f32)
                    + jnp.dot(xd, v1b, preferred_element_type=f32)
                    + vb_ref[0], 0.0)
    for l in range(3):
        g = jnp.maximum(jnp.dot(g, vw_ref[l], preferred_element_type=f32)
                        + vb_ref[l + 1], 0.0)
    m2_ref[...] = (jnp.dot(g, vw_ref[3], preferred_element_type=f32)
                   + vb_ref[4])


def _mid(xs, xd, ang, ew, eb, wfe, wfa, bf, vw1, vw, vb):
    e = xs.shape[0]
    return pl.pallas_call(
        _mid_body,
        grid=(e // BE,),
        in_specs=[_rows(BE, C), _rows(BE, C), _rows(BE, 1),
                  _full((5, 2 * C, 2 * C)), _full((5, 1, 2 * C)),
                  _full((2 * C, 2 * C)), _full((1, 2 * C)), _full((1, 2 * C)),
                  _full((2 * C, C)), _full((4, C, C)), _full((5, 1, C))],
        out_specs=[_rows(BE, 2 * C), _rows(BE, C),
                   pl.BlockSpec((1, 1), lambda i: (0, 0))],
        out_shape=[jax.ShapeDtypeStruct((e, 2 * C), jnp.float32),
                   jax.ShapeDtypeStruct((e, C), jnp.float32),
                   jax.ShapeDtypeStruct((1, 1), jnp.float32)],
    )(xs, xd, ang, ew, eb, wfe, wfa, bf, vw1, vw, vb)


def _ec2_body(xs_ref, xd_ref, e1_ref,
              uw1_ref, uw_ref, ub_ref, wfe_ref, wfi_ref, bf_ref,
              e2_ref, sl_ref):
    f32 = jnp.float32
    xs = xs_ref[...]
    xd = xd_ref[...]
    u1a = uw1_ref[:C]
    u1b = uw1_ref[C:]
    hij = jnp.maximum(jnp.dot(xs, u1a, preferred_element_type=f32)
                      + jnp.dot(xd, u1b, preferred_element_type=f32)
                      + ub_ref[0], 0.0)
    hji = jnp.maximum(jnp.dot(xd, u1a, preferred_element_type=f32)
                      + jnp.dot(xs, u1b, preferred_element_type=f32)
                      + ub_ref[0], 0.0)
    for l in range(3):
        w = uw_ref[l]
        hij = jnp.maximum(jnp.dot(hij, w, preferred_element_type=f32)
                          + ub_ref[l + 1], 0.0)
        hji = jnp.maximum(jnp.dot(hji, w, preferred_element_type=f32)
                          + ub_ref[l + 1], 0.0)
    w = uw_ref[3]
    fij = jnp.dot(hij, w, preferred_element_type=f32) + ub_ref[4]
    fji = jnp.dot(hji, w, preferred_element_type=f32) + ub_ref[4]
    d = fij - fji

    @pl.when(pl.program_id(0) == 0)
    def _():
        sl_ref[...] = jnp.zeros((1, 1), jnp.float32)

    sl_ref[...] += jnp.sum(d * d).reshape(1, 1)
    e = 0.5 * (fij + fji)
    e2_ref[...] = jnp.maximum(
        jnp.dot(e, wfe_ref[...], preferred_element_type=f32)
        + jnp.dot(e1_ref[...], wfi_ref[...], preferred_element_type=f32)
        + bf_ref[...], 0.0)


def _ec2(xs, xd, e1, uw1, uw, ub, wfe, wfi, bf):
    e = xs.shape[0]
    return pl.pallas_call(
        _ec2_body,
        grid=(e // BE,),
        in_specs=[_rows(BE, C), _rows(BE, C), _rows(BE, 2 * C),
                  _full((2 * C, C)), _full((4, C, C)), _full((5, 1, C)),
                  _full((C, C)), _full((2 * C, C)), _full((1, C))],
        out_specs=[_rows(BE, C), pl.BlockSpec((1, 1), lambda i: (0, 0))],
        out_shape=[jax.ShapeDtypeStruct((e, C), jnp.float32),
                   jax.ShapeDtypeStruct((1, 1), jnp.float32)],
    )(xs, xd, e1, uw1, uw, ub, wfe, wfi, bf)


def _stack_mlp(layers):
    w1 = layers[0][0]
    wt = jnp.stack([w for (w, _) in layers[1:]])
    bs = jnp.stack([b[None, :] for (_, b) in layers])
    return w1, wt, bs


def _pad_dst2d(dsth, e):
    epw, nch, rem = _geom(e)
    nrows = -(-(nch + (1 if rem else 0)) // 8) * 8
    dstw = dsth.reshape(SC_NW, epw)
    pad = jnp.full((SC_NW, nrows * CH - epw), NPAD - 1, jnp.int32)
    return jnp.concatenate([dstw, pad], axis=1).reshape(-1, CH)


def kernel(node_features, edge_index, angles, gt_edges,
           nc1, ec1_mlp, ec1_fuse, nc2, ec2_mlp, ec2_fuse):
    src, dst = edge_index[0], edge_index[1]
    x0 = node_features
    dst2d = _pad_dst2d(dst, N_EDGES)
    gath = _make_sc_gather(C, N_EDGES)
    gath2 = _make_sc_gather2(C, N_EDGES)
    seg = _make_sc_segsum(N_EDGES)
    cntp = _make_sc_count()(dst)

    n1w1, n1wt, n1b = _stack_mlp(nc1)
    ab1 = _node_mm(x0, jnp.concatenate([n1w1[:C], n1w1[C:]], axis=1))
    h1 = gath2(ab1[:, :C], ab1[:, C:], src, dst)
    s1p = seg(_nc_tail(h1, n1wt, n1b), dst2d)
    x1 = _node_update(x0, s1p, cntp)

    ew = jnp.stack([w for (w, _) in ec1_mlp])
    eb = jnp.stack([b[None, :] for (_, b) in ec1_mlp])
    wf1, bf1 = ec1_fuse
    n2w1, n2wt, n2b = _stack_mlp(nc2)
    xs1 = gath(x1, src)
    xd1 = gath(x1, dst)
    e1, m2, sl1s = _mid(xs1, xd1, angles, ew, eb,
                        wf1[:2 * C], wf1[2 * C:2 * C + 1], bf1[None, :],
                        n2w1, n2wt, n2b)
    s2p = seg(m2, dst2d)
    x2 = _node_update(x1, s2p, cntp)

    e2w1, e2wt, e2b = _stack_mlp(ec2_mlp)
    wf2, bf2 = ec2_fuse
    xs2 = gath(x2, src)
    xd2 = gath(x2, dst)
    e2, sl2s = _ec2(xs2, xd2, e1, e2w1, e2wt, e2b,
                    wf2[:C], wf2[C:], bf2[None, :])

    sl1 = sl1s[0, 0] / (N_EDGES * 2 * C)
    sl2 = sl2s[0, 0] / (N_EDGES * C)
    return (e2, (sl1 + sl2) / 2.0)
